# Initial kernel scaffold; baseline (speedup 1.0000x reference)
import jax
import jax.numpy as jnp
from jax.experimental import pallas as pl


def kernel(nxyz, nbr_list, num_atoms, embed, Wg1, bg1, Wg2, bg2, Win, bin, Wo1, bo1, Wo2, bo2, Wr1, br1, Wr2, br2, Wa, ba):
    raise NotImplementedError("write your pallas kernel here")


# speedup vs baseline: 4.8741x; 4.8741x over previous
import functools

import jax
import jax.numpy as jnp
from jax import lax
from jax.experimental import pallas as pl
from jax.experimental.pallas import tpu as pltpu
from jax.experimental.pallas import tpu_sc as plsc

N = 10000
E = 320000
B = 100
D = 128
G = 32
CONVS = 3
CUTOFF = 5.0

NP = 10240
NW = 32
C = 80
E_BLK = 3200
N_BLK = 2048

_LOG2 = 0.6931471805599453


def _ssp(x):
    return jnp.maximum(x, 0.0) + jnp.log1p(jnp.exp(-jnp.abs(x))) - _LOG2


def _sp(x):
    return jnp.maximum(x, 0.0) + jnp.log1p(jnp.exp(-jnp.abs(x)))


def _sig(x):
    return 1.0 / (1.0 + jnp.exp(-x))


def _mesh():
    return plsc.VectorSubcoreMesh(core_axis_name="c", subcore_axis_name="s")


@functools.lru_cache(maxsize=None)
def _sc_gather(n_idx, n_tab, dw):
    per_w = n_idx // NW
    nchunk = per_w // C

    @functools.partial(
        pl.kernel,
        mesh=_mesh(),
        out_type=jax.ShapeDtypeStruct((n_idx, dw), jnp.float32),
        compiler_params=pltpu.CompilerParams(use_tc_tiling_on_sc=False),
        scratch_types=[
            pltpu.VMEM((C,), jnp.int32),
            pltpu.VMEM((C, dw), jnp.float32),
            pltpu.SemaphoreType.DMA,
        ],
    )
    def k(table_hbm, idx_hbm, out_hbm, idx_v, rows_v, sem):
        wid = lax.axis_index("s") * 2 + lax.axis_index("c")
        base = wid * per_w

        def body(j, carry):
            off = base + j * C
            pltpu.sync_copy(idx_hbm.at[pl.ds(off, C)], idx_v)
            pltpu.async_copy(table_hbm.at[idx_v], rows_v, sem).wait()
            pltpu.sync_copy(rows_v, out_hbm.at[pl.ds(off, C)])
            return carry

        lax.fori_loop(0, nchunk, body, 0)

    return k


@functools.lru_cache(maxsize=None)
def _sc_scatter_add(n_vals, dw):
    per_w = n_vals // NW
    nchunk = per_w // C
    stripe = NP // 16

    @functools.partial(
        pl.kernel,
        mesh=_mesh(),
        out_type=jax.ShapeDtypeStruct((2, NP, dw), jnp.float32),
        compiler_params=pltpu.CompilerParams(use_tc_tiling_on_sc=False),
        scratch_types=[
            pltpu.VMEM((C,), jnp.int32),
            pltpu.VMEM((C, dw), jnp.float32),
            pltpu.VMEM_SHARED((NP, dw), jnp.float32),
            pltpu.SemaphoreType.DMA,
        ],
    )
    def k(vals_hbm, idx_hbm, zrow_hbm, out_hbm, idx_v, vals_v, acc_sh, sem):
        cid = lax.axis_index("c")
        sid = lax.axis_index("s")
        wid = sid * 2 + cid
        base = wid * per_w

        pltpu.sync_copy(zrow_hbm, vals_v)

        def zbody(j, carry):
            pltpu.sync_copy(vals_v, acc_sh.at[pl.ds(sid * stripe + j * C, C)])
            return carry

        lax.fori_loop(0, stripe // C, zbody, 0)
        plsc.subcore_barrier()

        def body(j, carry):
            off = base + j * C
            pltpu.sync_copy(idx_hbm.at[pl.ds(off, C)], idx_v)
            pltpu.sync_copy(vals_hbm.at[pl.ds(off, C)], vals_v)
            pltpu.sync_copy(vals_v, acc_sh.at[idx_v], add=True)
            return carry

        lax.fori_loop(0, nchunk, body, 0)
        plsc.subcore_barrier()
        pltpu.sync_copy(
            acc_sh.at[pl.ds(sid * stripe, stripe)],
            out_hbm.at[cid, pl.ds(sid * stripe, stripe)],
        )

    return k


def _tc_map(fn, rows, auxs, out_cols, blk):
    R = rows[0].shape[0]
    grid = R // blk

    def row_spec(x):
        nd = x.ndim
        bs = (blk,) + x.shape[1:]
        return pl.BlockSpec(bs, lambda i, nd=nd: (i,) + (0,) * (nd - 1))

    def aux_spec(a):
        nd = a.ndim
        return pl.BlockSpec(a.shape, lambda i, nd=nd: (0,) * nd)

    in_specs = [row_spec(x) for x in rows] + [aux_spec(a) for a in auxs]
    out_specs = [pl.BlockSpec((blk, c), lambda i: (i, 0)) for c in out_cols]
    out_shape = [jax.ShapeDtypeStruct((R, c), jnp.float32) for c in out_cols]
    n_in = len(rows) + len(auxs)

    def body(*refs):
        vals = fn(*[ref[...] for ref in refs[:n_in]])
        if not isinstance(vals, (tuple, list)):
            vals = (vals,)
        for oref, v in zip(refs[n_in:], vals):
            oref[...] = v

    res = pl.pallas_call(
        body, grid=(grid,), in_specs=in_specs, out_specs=out_specs,
        out_shape=out_shape,
    )(*rows, *auxs)
    return res if len(out_cols) > 1 else res[0]


def _tc_whole(fn, ins, out_shapes):
    in_specs = [pl.BlockSpec(x.shape, lambda *_, nd=x.ndim: (0,) * nd) for x in ins]
    out_specs = [pl.BlockSpec(s, lambda *_, nd=len(s): (0,) * nd) for s in out_shapes]
    out_shape = [jax.ShapeDtypeStruct(s, jnp.float32) for s in out_shapes]
    n_in = len(ins)

    def body(*refs):
        vals = fn(*[ref[...] for ref in refs[:n_in]])
        if not isinstance(vals, (tuple, list)):
            vals = (vals,)
        for oref, v in zip(refs[n_in:], vals):
            oref[...] = v

    res = pl.pallas_call(
        body, in_specs=in_specs, out_specs=out_specs, out_shape=out_shape,
    )(*ins)
    return res if len(out_shapes) > 1 else res[0]


def kernel(nxyz, nbr_list, num_atoms, embed, Wg1, bg1, Wg2, bg2, Win, bin,
           Wo1, bo1, Wo2, bo2, Wr1, br1, Wr2, br2, Wa, ba):
    f32 = jnp.float32
    sigma = float(CUTOFF / (G - 1))
    inv2s2 = 1.0 / (2.0 * sigma * sigma)

    def _mu_row():
        return lax.broadcasted_iota(jnp.int32, (1, G), 1).astype(f32) * (
            CUTOFF / (G - 1))

    z = nxyz[:, 0].astype(jnp.int32)
    xyz = nxyz[:, 1:4].astype(f32)
    a0 = nbr_list[:, 0].astype(jnp.int32)
    a1 = nbr_list[:, 1].astype(jnp.int32)

    zp = jnp.pad(z, (0, NP - N))
    xyzp = jnp.pad(xyz, ((0, NP - N), (0, 13)))
    zrow16 = jnp.zeros((C, 16), f32)
    zrow128 = jnp.zeros((C, D), f32)

    r = _sc_gather(NP, 100, D)(embed, zp)
    x0 = _sc_gather(E, NP, 16)(xyzp, a0)
    x1 = _sc_gather(E, NP, 16)(xyzp, a1)

    def geom_fn(x0b, x1b):
        df = x0b - x1b
        d = jnp.sqrt(jnp.sum(df * df, axis=1, keepdims=True) + 1e-12)
        gv = jnp.exp(-((d - _mu_row()) ** 2) * inv2s2)
        return d, gv

    dcol, gvec = _tc_map(geom_fn, [x0, x1], [], [1, G], E_BLK)

    b2 = lambda v: v.reshape(1, -1)
    Ws_l, h1_l, u_l = [], [], []
    for i in range(CONVS):
        def filt_fn(gb, W1, b1, W2, bb2):
            return _ssp(jnp.dot(gb, W1, preferred_element_type=f32) + b1) @ W2 + bb2

        We = _tc_map(filt_fn, [gvec], [Wg1[i], b2(bg1[i]), Wg2[i], b2(bg2[i])],
                     [D], E_BLK)

        def h_fn(rb, W, bb):
            return jnp.dot(rb, W, preferred_element_type=f32) + bb

        h = _tc_map(h_fn, [r], [Win[i], b2(bin[i])], [D], N_BLK)
        h1 = _sc_gather(E, NP, D)(h, a1)
        m = _tc_map(lambda a, b: a * b, [h1, We], [], [D], E_BLK)
        aggp = _sc_scatter_add(E, D)(m, a0, zrow128)

        def out_fn(rb, g0b, g1b, W1, b1, W2, bb2):
            u = jnp.dot(g0b + g1b, W1, preferred_element_type=f32) + b1
            rn = rb + jnp.dot(_ssp(u), W2, preferred_element_type=f32) + bb2
            return u, rn

        u, r = _tc_map(out_fn, [r, aggp[0], aggp[1]],
                       [Wo1[i], b2(bo1[i]), Wo2[i], b2(bo2[i])], [D, D], N_BLK)
        Ws_l.append(We)
        h1_l.append(h1)
        u_l.append(u)

    Wa8 = jnp.pad(Wa, ((0, 0), (0, 0), (0, 5)))
    ba8 = jnp.pad(ba, ((0, 0), (0, 5)))
    ur_l, p_l, ae_l, pa_l = [], [], [], []
    for k in range(2):
        def ro_fn(rb, W1, b1, W2, bb2, Wap, bap):
            ur = jnp.dot(rb, W1, preferred_element_type=f32) + b1
            ae = jnp.dot(_ssp(ur), W2, preferred_element_type=f32) + bb2
            p = jnp.dot(rb, Wap, preferred_element_type=f32) + bap
            return ur, ae, p, _sp(p)

        ur, ae, p, pa = _tc_map(
            ro_fn, [r],
            [Wr1[k], b2(br1[k]), Wr2[k], b2(br2[k]), Wa8[k], b2(ba8[k])],
            [64, 1, 8, 8], N_BLK)
        ur_l.append(ur); p_l.append(p); ae_l.append(ae); pa_l.append(pa)

    patom = jnp.concatenate(
        [pa_l[0][:, :3], pa_l[1][:, :3], jnp.zeros((NP, 10), f32)], axis=1)
    P0 = _sc_gather(E, NP, 16)(patom, a0)
    P1 = _sc_gather(E, NP, 16)(patom, a1)

    def eb_fn(p0b, p1b, db):
        cols = []
        d1 = db[:, 0]
        for k in range(2):
            Dp = 0.5 * (p0b[:, 3 * k] + p1b[:, 3 * k])
            ap = 0.5 * (p0b[:, 3 * k + 1] + p1b[:, 3 * k + 1])
            r0 = 0.5 * (p0b[:, 3 * k + 2] + p1b[:, 3 * k + 2])
            q = 1.0 - jnp.exp(-ap * (d1 - r0))
            cols.append((Dp * q * q)[:, None])
        return jnp.concatenate(cols + [jnp.zeros_like(p0b[:, :14])], axis=1)

    ebrow = _tc_map(eb_fn, [P0, P1, dcol], [], [16], E_BLK)
    ebp = _sc_scatter_add(E, 16)(ebrow, a0, zrow16)

    def mol_fn(ae0b, ae1b, e00, e01, e10, e11):
        m0 = jnp.sum(ae0b, axis=1) + jnp.sum(e00 + e01, axis=1)
        m1 = jnp.sum(ae1b, axis=1) + jnp.sum(e10 + e11, axis=1)
        se0 = jnp.minimum(m0, m1)
        se1 = jnp.maximum(m0, m1)
        t0 = (m0 <= m1).astype(f32)
        out = jnp.stack([se0, se1, t0, 1.0 - t0], axis=1)
        return jnp.pad(out, ((0, 0), (0, 4)))

    mol_ins = [ae_l[0][:N, 0].reshape(B, N // B), ae_l[1][:N, 0].reshape(B, N // B),
               ebp[0, :N, 0].reshape(B, N // B), ebp[1, :N, 0].reshape(B, N // B),
               ebp[0, :N, 1].reshape(B, N // B), ebp[1, :N, 1].reshape(B, N // B)]
    mol = _tc_whole(mol_fn, mol_ins, [(B, 8)])
    se0, se1 = mol[:, 0], mol[:, 1]
    w = mol[:, 2:4]
    seeds_at = jnp.repeat(
        jnp.concatenate([w, 1.0 - w], axis=1), N // B, axis=0)
    seeds_at = jnp.pad(seeds_at, ((0, NP - N), (0, 0)))

    patom2 = jnp.concatenate(
        [pa_l[0][:, :3], pa_l[1][:, :3], seeds_at, jnp.zeros((NP, 6), f32)],
        axis=1)
    P0b = _sc_gather(E, NP, 16)(patom2, a0)
    P1b = _sc_gather(E, NP, 16)(patom2, a1)

    Wr2T = [b2(Wr2[k][:, 0]) for k in range(2)]
    Wr1T = [Wr1[k].T for k in range(2)]
    WaT8 = [jnp.pad(Wa[k].T, ((0, 5), (0, 0))) for k in range(2)]
    Wo2T = [Wo2[i].T for i in range(CONVS)]
    Wo1T = [Wo1[i].T for i in range(CONVS)]
    WinT = [Win[i].T for i in range(CONVS)]
    Wg2T = [Wg2[i].T for i in range(CONVS)]
    Wg1T = [Wg1[i].T for i in range(CONVS)]

    def backward(c):
        def ebb_fn(p0b, p1b, db):
            d1 = db[:, 0]
            dbar = jnp.zeros_like(d1)
            cols = []
            for k in range(2):
                seed = p0b[:, 6 + 2 * c + k]
                Dp = 0.5 * (p0b[:, 3 * k] + p1b[:, 3 * k])
                ap = 0.5 * (p0b[:, 3 * k + 1] + p1b[:, 3 * k + 1])
                r0 = 0.5 * (p0b[:, 3 * k + 2] + p1b[:, 3 * k + 2])
                ex = jnp.exp(-ap * (d1 - r0))
                q = 1.0 - ex
                Dpb = seed * q * q
                qb = seed * Dp * 2.0 * q
                apb = qb * ex * (d1 - r0)
                dbar = dbar + qb * ex * ap
                r0b = -qb * ex * ap
                cols += [(0.5 * Dpb)[:, None], (0.5 * apb)[:, None],
                         (0.5 * r0b)[:, None]]
            crow = jnp.concatenate(cols + [jnp.zeros_like(p0b[:, :10])], axis=1)
            return dbar[:, None], crow

        dbar_eb, crow = _tc_map(ebb_fn, [P0b, P1b, dcol], [], [1, 16], E_BLK)
        pb0 = _sc_scatter_add(E, 16)(crow, a0, zrow16)
        pb1 = _sc_scatter_add(E, 16)(crow, a1, zrow16)

        def rbar_fn(pba, pbb, pbc, pbd, ur0, ur1, pp0, pp1, sa,
                    wr2t0, wr1t0, wat0, wr2t1, wr1t1, wat1):
            pb = pba + pbb + pbc + pbd
            rb = jnp.zeros((pba.shape[0], D), f32)
            for k, (wr2t, wr1t, wat, urk, ppk) in enumerate(
                    [(wr2t0, wr1t0, wat0, ur0, pp0),
                     (wr2t1, wr1t1, wat1, ur1, pp1)]):
                seed = sa[:, 2 * c + k:2 * c + k + 1]
                urb = (seed * wr2t) * _sig(urk)
                rb = rb + jnp.dot(urb, wr1t, preferred_element_type=f32)
                pbk = pb[:, 3 * k:3 * k + 3] * _sig(ppk[:, :3])
                pbk = jnp.concatenate(
                    [pbk, jnp.zeros_like(ppk[:, :5])], axis=1)
                rb = rb + jnp.dot(pbk, wat, preferred_element_type=f32)
            return rb

        sa_cols = jnp.concatenate([seeds_at, jnp.zeros((NP, 4), f32)], axis=1)
        rbar = _tc_map(
            rbar_fn,
            [pb0[0], pb0[1], pb1[0], pb1[1], ur_l[0], ur_l[1],
             p_l[0], p_l[1], sa_cols],
            [Wr2T[0], Wr1T[0], WaT8[0], Wr2T[1], Wr1T[1], WaT8[1]],
            [D], N_BLK)

        gbar = None
        for i in reversed(range(CONVS)):
            def aggb_fn(rbb, ub, w2t, w1t):
                vb = jnp.dot(rbb, w2t, preferred_element_type=f32)
                return jnp.dot(vb * _sig(ub), w1t, preferred_element_type=f32)

            aggbar = _tc_map(aggb_fn, [rbar, u_l[i]], [Wo2T[i], Wo1T[i]],
                             [D], N_BLK)
            mb = _sc_gather(E, NP, D)(aggbar, a0)

            def wb_fn(mbb, h1b, web):
                return mbb * h1b, mbb * web

            Wbar, hbrow = _tc_map(wb_fn, [mb, h1_l[i], Ws_l[i]], [],
                                  [D, D], E_BLK)
            hbp = _sc_scatter_add(E, D)(hbrow, a1, zrow128)

            def rbup_fn(rbb, hb0, hb1, wt):
                return rbb + jnp.dot(hb0 + hb1, wt, preferred_element_type=f32)

            rbar = _tc_map(rbup_fn, [rbar, hbp[0], hbp[1]], [WinT[i]],
                           [D], N_BLK)

            def gb_fn(wbb, gvb, w2t, w1, b1, w1t, *gprev):
                sb = jnp.dot(wbb, w2t, preferred_element_type=f32)
                tb = sb * _sig(jnp.dot(gvb, w1, preferred_element_type=f32) + b1)
                out = jnp.dot(tb, w1t, preferred_element_type=f32)
                return out + gprev[0] if gprev else out

            gins = [Wbar, gvec] + ([gbar] if gbar is not None else [])

            def gb_wrap(wbb, gvb, *rest):
                auxs = rest[-4:]
                gp = rest[:-4]
                return gb_fn(wbb, gvb, auxs[0], auxs[1], auxs[2], auxs[3], *gp)

            gbar = _tc_map(gb_wrap, gins,
                           [Wg2T[i], Wg1[i], b2(bg1[i]), Wg1T[i]], [G], E_BLK)

        def delta_fn(dbe, gbb, gvb, db, x0b, x1b):
            d1 = db[:, 0:1]
            dbar = dbe + jnp.sum(
                gbb * gvb * (_mu_row() - d1), axis=1, keepdims=True) * (
                    1.0 / (sigma * sigma))
            coef = dbar / d1
            delta = coef * (x0b - x1b)
            return delta, -delta

        delta, ndelta = _tc_map(delta_fn, [dbar_eb, gbar, gvec, dcol, x0, x1],
                                [], [16, 16], E_BLK)
        gp0 = _sc_scatter_add(E, 16)(delta, a0, zrow16)
        gp1 = _sc_scatter_add(E, 16)(ndelta, a1, zrow16)
        gsum = _tc_map(lambda a, bq, cq, dq: a + bq + cq + dq,
                       [gp0[0], gp0[1], gp1[0], gp1[1]], [], [16], N_BLK)
        return gsum[:N, :3]

    g0 = backward(0)
    g1 = backward(1)
    return (se0, se1, g0, g1)


# speedup vs baseline: 5.1726x; 1.0612x over previous
import functools

import jax
import jax.numpy as jnp
from jax import lax
from jax.experimental import pallas as pl
from jax.experimental.pallas import tpu as pltpu
from jax.experimental.pallas import tpu_sc as plsc

N = 10000
E = 320000
B = 100
D = 128
G = 32
CONVS = 3
CUTOFF = 5.0

NP = 10240
NW = 32
C = 80
E_BLK = 3200
N_BLK = 2048

_LOG2 = 0.6931471805599453


def _ssp(x):
    return jnp.maximum(x, 0.0) + jnp.log1p(jnp.exp(-jnp.abs(x))) - _LOG2


def _sp(x):
    return jnp.maximum(x, 0.0) + jnp.log1p(jnp.exp(-jnp.abs(x)))


def _sig(x):
    return 1.0 / (1.0 + jnp.exp(-x))


def _mesh():
    return plsc.VectorSubcoreMesh(core_axis_name="c", subcore_axis_name="s")


@functools.lru_cache(maxsize=None)
def _sc_gather(n_idx, n_tab, dw):
    per_w = n_idx // NW
    nchunk = per_w // C

    @functools.partial(
        pl.kernel,
        mesh=_mesh(),
        out_type=jax.ShapeDtypeStruct((n_idx, dw), jnp.float32),
        compiler_params=pltpu.CompilerParams(use_tc_tiling_on_sc=False),
        scratch_types=[
            pltpu.VMEM((C,), jnp.int32),
            pltpu.VMEM((C, dw), jnp.float32),
            pltpu.SemaphoreType.DMA,
        ],
    )
    def k(table_hbm, idx_hbm, out_hbm, idx_v, rows_v, sem):
        wid = lax.axis_index("s") * 2 + lax.axis_index("c")
        base = wid * per_w

        def body(j, carry):
            off = base + j * C
            pltpu.sync_copy(idx_hbm.at[pl.ds(off, C)], idx_v)
            pltpu.async_copy(table_hbm.at[idx_v], rows_v, sem).wait()
            pltpu.sync_copy(rows_v, out_hbm.at[pl.ds(off, C)])
            return carry

        lax.fori_loop(0, nchunk, body, 0)

    return k


@functools.lru_cache(maxsize=None)
def _sc_scatter_add(n_vals, dw):
    per_w = n_vals // NW
    nchunk = per_w // C
    stripe = NP // 16

    @functools.partial(
        pl.kernel,
        mesh=_mesh(),
        out_type=jax.ShapeDtypeStruct((2, NP, dw), jnp.float32),
        compiler_params=pltpu.CompilerParams(use_tc_tiling_on_sc=False),
        scratch_types=[
            pltpu.VMEM((C,), jnp.int32),
            pltpu.VMEM((C, dw), jnp.float32),
            pltpu.VMEM_SHARED((NP, dw), jnp.float32),
            pltpu.SemaphoreType.DMA,
        ],
    )
    def k(vals_hbm, idx_hbm, zrow_hbm, out_hbm, idx_v, vals_v, acc_sh, sem):
        cid = lax.axis_index("c")
        sid = lax.axis_index("s")
        wid = sid * 2 + cid
        base = wid * per_w

        pltpu.sync_copy(zrow_hbm, vals_v)

        def zbody(j, carry):
            pltpu.sync_copy(vals_v, acc_sh.at[pl.ds(sid * stripe + j * C, C)])
            return carry

        lax.fori_loop(0, stripe // C, zbody, 0)
        plsc.subcore_barrier()

        def body(j, carry):
            off = base + j * C
            pltpu.sync_copy(idx_hbm.at[pl.ds(off, C)], idx_v)
            pltpu.sync_copy(vals_hbm.at[pl.ds(off, C)], vals_v)
            pltpu.sync_copy(vals_v, acc_sh.at[idx_v], add=True)
            return carry

        lax.fori_loop(0, nchunk, body, 0)
        plsc.subcore_barrier()
        pltpu.sync_copy(
            acc_sh.at[pl.ds(sid * stripe, stripe)],
            out_hbm.at[cid, pl.ds(sid * stripe, stripe)],
        )

    return k


@functools.lru_cache(maxsize=None)
def _sc_conv_fwd():
    per_w = E // NW
    nchunk = per_w // C
    stripe = NP // 16

    @functools.partial(
        pl.kernel,
        mesh=_mesh(),
        out_type=jax.ShapeDtypeStruct((2, NP, D), jnp.float32),
        scratch_types=[
            pltpu.VMEM((C,), jnp.int32),
            pltpu.VMEM((C,), jnp.int32),
            pltpu.VMEM((C, D), jnp.float32),
            pltpu.VMEM((C, D), jnp.float32),
            pltpu.VMEM_SHARED((NP, D), jnp.float32),
            pltpu.SemaphoreType.DMA,
        ],
    )
    def k(h_hbm, we_hbm, a0_hbm, a1_hbm, zrow_hbm, out_hbm,
          idx0_v, idx1_v, hrow_v, werow_v, acc_sh, sem):
        cid = lax.axis_index("c")
        sid = lax.axis_index("s")
        base = (sid * 2 + cid) * per_w

        pltpu.sync_copy(zrow_hbm, hrow_v)

        def zbody(j, carry):
            pltpu.sync_copy(hrow_v, acc_sh.at[pl.ds(sid * stripe + j * C, C)])
            return carry

        lax.fori_loop(0, stripe // C, zbody, 0)
        plsc.subcore_barrier()

        def body(j, carry):
            off = base + j * C
            pltpu.sync_copy(a1_hbm.at[pl.ds(off, C)], idx1_v)
            pltpu.async_copy(h_hbm.at[idx1_v], hrow_v, sem).wait()
            pltpu.sync_copy(we_hbm.at[pl.ds(off, C)], werow_v)
            pltpu.sync_copy(a0_hbm.at[pl.ds(off, C)], idx0_v)

            def mul(i, carry2):
                for v in range(D // 16):
                    s = pl.ds(v * 16, 16)
                    hrow_v[i, s] = hrow_v[i, s] * werow_v[i, s]
                return carry2

            lax.fori_loop(0, C, mul, 0)
            pltpu.sync_copy(hrow_v, acc_sh.at[idx0_v], add=True)
            return carry

        lax.fori_loop(0, nchunk, body, 0)
        plsc.subcore_barrier()
        pltpu.sync_copy(
            acc_sh.at[pl.ds(sid * stripe, stripe)],
            out_hbm.at[cid, pl.ds(sid * stripe, stripe)],
        )

    return k


@functools.lru_cache(maxsize=None)
def _sc_conv_bwd():
    per_w = E // NW
    nchunk = per_w // C
    stripe = NP // 16

    @functools.partial(
        pl.kernel,
        mesh=_mesh(),
        out_type=(jax.ShapeDtypeStruct((E, D), jnp.float32),
                  jax.ShapeDtypeStruct((2, NP, D), jnp.float32)),
        scratch_types=[
            pltpu.VMEM((C,), jnp.int32),
            pltpu.VMEM((C,), jnp.int32),
            pltpu.VMEM((C, D), jnp.float32),
            pltpu.VMEM((C, D), jnp.float32),
            pltpu.VMEM((C, D), jnp.float32),
            pltpu.VMEM_SHARED((NP, D), jnp.float32),
            pltpu.SemaphoreType.DMA,
            pltpu.SemaphoreType.DMA,
        ],
    )
    def k(ab_hbm, h_hbm, we_hbm, a0_hbm, a1_hbm, zrow_hbm, wbar_hbm, out_hbm,
          idx0_v, idx1_v, mbrow_v, hrow_v, werow_v, acc_sh, sem, sem2):
        cid = lax.axis_index("c")
        sid = lax.axis_index("s")
        base = (sid * 2 + cid) * per_w

        pltpu.sync_copy(zrow_hbm, hrow_v)

        def zbody(j, carry):
            pltpu.sync_copy(hrow_v, acc_sh.at[pl.ds(sid * stripe + j * C, C)])
            return carry

        lax.fori_loop(0, stripe // C, zbody, 0)
        plsc.subcore_barrier()

        def body(j, carry):
            off = base + j * C
            pltpu.sync_copy(a0_hbm.at[pl.ds(off, C)], idx0_v)
            pltpu.sync_copy(a1_hbm.at[pl.ds(off, C)], idx1_v)
            cp1 = pltpu.async_copy(ab_hbm.at[idx0_v], mbrow_v, sem)
            cp2 = pltpu.async_copy(h_hbm.at[idx1_v], hrow_v, sem2)
            pltpu.sync_copy(we_hbm.at[pl.ds(off, C)], werow_v)
            cp1.wait()
            cp2.wait()

            def mul(i, carry2):
                for v in range(D // 16):
                    s = pl.ds(v * 16, 16)
                    mb = mbrow_v[i, s]
                    hrow_v[i, s] = mb * hrow_v[i, s]
                    werow_v[i, s] = mb * werow_v[i, s]
                return carry2

            lax.fori_loop(0, C, mul, 0)
            pltpu.sync_copy(hrow_v, wbar_hbm.at[pl.ds(off, C)])
            pltpu.sync_copy(werow_v, acc_sh.at[idx1_v], add=True)
            return carry

        lax.fori_loop(0, nchunk, body, 0)
        plsc.subcore_barrier()
        pltpu.sync_copy(
            acc_sh.at[pl.ds(sid * stripe, stripe)],
            out_hbm.at[cid, pl.ds(sid * stripe, stripe)],
        )

    return k


def _tc_map(fn, rows, auxs, out_cols, blk):
    R = rows[0].shape[0]
    grid = R // blk

    def row_spec(x):
        nd = x.ndim
        bs = (blk,) + x.shape[1:]
        return pl.BlockSpec(bs, lambda i, nd=nd: (i,) + (0,) * (nd - 1))

    def aux_spec(a):
        nd = a.ndim
        return pl.BlockSpec(a.shape, lambda i, nd=nd: (0,) * nd)

    in_specs = [row_spec(x) for x in rows] + [aux_spec(a) for a in auxs]
    out_specs = [pl.BlockSpec((blk, c), lambda i: (i, 0)) for c in out_cols]
    out_shape = [jax.ShapeDtypeStruct((R, c), jnp.float32) for c in out_cols]
    n_in = len(rows) + len(auxs)

    def body(*refs):
        vals = fn(*[ref[...] for ref in refs[:n_in]])
        if not isinstance(vals, (tuple, list)):
            vals = (vals,)
        for oref, v in zip(refs[n_in:], vals):
            oref[...] = v

    res = pl.pallas_call(
        body, grid=(grid,), in_specs=in_specs, out_specs=out_specs,
        out_shape=out_shape,
    )(*rows, *auxs)
    return res if len(out_cols) > 1 else res[0]


def _tc_whole(fn, ins, out_shapes):
    in_specs = [pl.BlockSpec(x.shape, lambda *_, nd=x.ndim: (0,) * nd) for x in ins]
    out_specs = [pl.BlockSpec(s, lambda *_, nd=len(s): (0,) * nd) for s in out_shapes]
    out_shape = [jax.ShapeDtypeStruct(s, jnp.float32) for s in out_shapes]
    n_in = len(ins)

    def body(*refs):
        vals = fn(*[ref[...] for ref in refs[:n_in]])
        if not isinstance(vals, (tuple, list)):
            vals = (vals,)
        for oref, v in zip(refs[n_in:], vals):
            oref[...] = v

    res = pl.pallas_call(
        body, in_specs=in_specs, out_specs=out_specs, out_shape=out_shape,
    )(*ins)
    return res if len(out_shapes) > 1 else res[0]


def kernel(nxyz, nbr_list, num_atoms, embed, Wg1, bg1, Wg2, bg2, Win, bin,
           Wo1, bo1, Wo2, bo2, Wr1, br1, Wr2, br2, Wa, ba):
    f32 = jnp.float32
    sigma = float(CUTOFF / (G - 1))
    inv2s2 = 1.0 / (2.0 * sigma * sigma)

    def _mu_row():
        return lax.broadcasted_iota(jnp.int32, (1, G), 1).astype(f32) * (
            CUTOFF / (G - 1))

    z = nxyz[:, 0].astype(jnp.int32)
    xyz = nxyz[:, 1:4].astype(f32)
    a0 = nbr_list[:, 0].astype(jnp.int32)
    a1 = nbr_list[:, 1].astype(jnp.int32)

    zp = jnp.pad(z, (0, NP - N))
    xyzp = jnp.pad(xyz, ((0, NP - N), (0, 13)))
    zrow16 = jnp.zeros((C, 16), f32)
    zrow128 = jnp.zeros((C, D), f32)

    r = _sc_gather(NP, 100, D)(embed, zp)
    x0 = _sc_gather(E, NP, 16)(xyzp, a0)
    x1 = _sc_gather(E, NP, 16)(xyzp, a1)

    def geom_fn(x0b, x1b):
        df = x0b - x1b
        d = jnp.sqrt(jnp.sum(df * df, axis=1, keepdims=True) + 1e-12)
        gv = jnp.exp(-((d - _mu_row()) ** 2) * inv2s2)
        return d, gv

    dcol, gvec = _tc_map(geom_fn, [x0, x1], [], [1, G], E_BLK)

    b2 = lambda v: v.reshape(1, -1)
    Ws_l, h_l, u_l = [], [], []
    for i in range(CONVS):
        def filt_fn(gb, W1, b1, W2, bb2):
            return _ssp(jnp.dot(gb, W1, preferred_element_type=f32) + b1) @ W2 + bb2

        We = _tc_map(filt_fn, [gvec], [Wg1[i], b2(bg1[i]), Wg2[i], b2(bg2[i])],
                     [D], E_BLK)

        def h_fn(rb, W, bb):
            return jnp.dot(rb, W, preferred_element_type=f32) + bb

        h = _tc_map(h_fn, [r], [Win[i], b2(bin[i])], [D], N_BLK)
        aggp = _sc_conv_fwd()(h, We, a0, a1, zrow128)

        def out_fn(rb, g0b, g1b, W1, b1, W2, bb2):
            u = jnp.dot(g0b + g1b, W1, preferred_element_type=f32) + b1
            rn = rb + jnp.dot(_ssp(u), W2, preferred_element_type=f32) + bb2
            return u, rn

        u, r = _tc_map(out_fn, [r, aggp[0], aggp[1]],
                       [Wo1[i], b2(bo1[i]), Wo2[i], b2(bo2[i])], [D, D], N_BLK)
        Ws_l.append(We)
        h_l.append(h)
        u_l.append(u)

    Wa8 = jnp.pad(Wa, ((0, 0), (0, 0), (0, 5)))
    ba8 = jnp.pad(ba, ((0, 0), (0, 5)))
    ur_l, p_l, ae_l, pa_l = [], [], [], []
    for k in range(2):
        def ro_fn(rb, W1, b1, W2, bb2, Wap, bap):
            ur = jnp.dot(rb, W1, preferred_element_type=f32) + b1
            ae = jnp.dot(_ssp(ur), W2, preferred_element_type=f32) + bb2
            p = jnp.dot(rb, Wap, preferred_element_type=f32) + bap
            return ur, ae, p, _sp(p)

        ur, ae, p, pa = _tc_map(
            ro_fn, [r],
            [Wr1[k], b2(br1[k]), Wr2[k], b2(br2[k]), Wa8[k], b2(ba8[k])],
            [64, 1, 8, 8], N_BLK)
        ur_l.append(ur); p_l.append(p); ae_l.append(ae); pa_l.append(pa)

    patom = jnp.concatenate(
        [pa_l[0][:, :3], pa_l[1][:, :3], jnp.zeros((NP, 10), f32)], axis=1)
    P0 = _sc_gather(E, NP, 16)(patom, a0)
    P1 = _sc_gather(E, NP, 16)(patom, a1)

    def eb_fn(p0b, p1b, db):
        cols = []
        d1 = db[:, 0]
        for k in range(2):
            Dp = 0.5 * (p0b[:, 3 * k] + p1b[:, 3 * k])
            ap = 0.5 * (p0b[:, 3 * k + 1] + p1b[:, 3 * k + 1])
            r0 = 0.5 * (p0b[:, 3 * k + 2] + p1b[:, 3 * k + 2])
            q = 1.0 - jnp.exp(-ap * (d1 - r0))
            cols.append((Dp * q * q)[:, None])
        return jnp.concatenate(cols + [jnp.zeros_like(p0b[:, :14])], axis=1)

    ebrow = _tc_map(eb_fn, [P0, P1, dcol], [], [16], E_BLK)
    ebp = _sc_scatter_add(E, 16)(ebrow, a0, zrow16)

    def mol_fn(ae0b, ae1b, e00, e01, e10, e11):
        m0 = jnp.sum(ae0b, axis=1) + jnp.sum(e00 + e01, axis=1)
        m1 = jnp.sum(ae1b, axis=1) + jnp.sum(e10 + e11, axis=1)
        se0 = jnp.minimum(m0, m1)
        se1 = jnp.maximum(m0, m1)
        t0 = (m0 <= m1).astype(f32)
        out = jnp.stack([se0, se1, t0, 1.0 - t0], axis=1)
        return jnp.pad(out, ((0, 0), (0, 4)))

    mol_ins = [ae_l[0][:N, 0].reshape(B, N // B), ae_l[1][:N, 0].reshape(B, N // B),
               ebp[0, :N, 0].reshape(B, N // B), ebp[1, :N, 0].reshape(B, N // B),
               ebp[0, :N, 1].reshape(B, N // B), ebp[1, :N, 1].reshape(B, N // B)]
    mol = _tc_whole(mol_fn, mol_ins, [(B, 8)])
    se0, se1 = mol[:, 0], mol[:, 1]
    w = mol[:, 2:4]
    seeds_at = jnp.repeat(
        jnp.concatenate([w, 1.0 - w], axis=1), N // B, axis=0)
    seeds_at = jnp.pad(seeds_at, ((0, NP - N), (0, 0)))

    patom2 = jnp.concatenate(
        [pa_l[0][:, :3], pa_l[1][:, :3], seeds_at, jnp.zeros((NP, 6), f32)],
        axis=1)
    P0b = _sc_gather(E, NP, 16)(patom2, a0)
    P1b = _sc_gather(E, NP, 16)(patom2, a1)

    Wr2T = [b2(Wr2[k][:, 0]) for k in range(2)]
    Wr1T = [Wr1[k].T for k in range(2)]
    WaT8 = [jnp.pad(Wa[k].T, ((0, 5), (0, 0))) for k in range(2)]
    Wo2T = [Wo2[i].T for i in range(CONVS)]
    Wo1T = [Wo1[i].T for i in range(CONVS)]
    WinT = [Win[i].T for i in range(CONVS)]
    Wg2T = [Wg2[i].T for i in range(CONVS)]
    Wg1T = [Wg1[i].T for i in range(CONVS)]

    def backward(c):
        def ebb_fn(p0b, p1b, db):
            d1 = db[:, 0]
            dbar = jnp.zeros_like(d1)
            cols = []
            for k in range(2):
                seed = p0b[:, 6 + 2 * c + k]
                Dp = 0.5 * (p0b[:, 3 * k] + p1b[:, 3 * k])
                ap = 0.5 * (p0b[:, 3 * k + 1] + p1b[:, 3 * k + 1])
                r0 = 0.5 * (p0b[:, 3 * k + 2] + p1b[:, 3 * k + 2])
                ex = jnp.exp(-ap * (d1 - r0))
                q = 1.0 - ex
                Dpb = seed * q * q
                qb = seed * Dp * 2.0 * q
                apb = qb * ex * (d1 - r0)
                dbar = dbar + qb * ex * ap
                r0b = -qb * ex * ap
                cols += [(0.5 * Dpb)[:, None], (0.5 * apb)[:, None],
                         (0.5 * r0b)[:, None]]
            crow = jnp.concatenate(cols + [jnp.zeros_like(p0b[:, :10])], axis=1)
            return dbar[:, None], crow

        dbar_eb, crow = _tc_map(ebb_fn, [P0b, P1b, dcol], [], [1, 16], E_BLK)
        pb0 = _sc_scatter_add(E, 16)(crow, a0, zrow16)
        pb1 = _sc_scatter_add(E, 16)(crow, a1, zrow16)

        def rbar_fn(pba, pbb, pbc, pbd, ur0, ur1, pp0, pp1, sa,
                    wr2t0, wr1t0, wat0, wr2t1, wr1t1, wat1):
            pb = pba + pbb + pbc + pbd
            rb = jnp.zeros((pba.shape[0], D), f32)
            for k, (wr2t, wr1t, wat, urk, ppk) in enumerate(
                    [(wr2t0, wr1t0, wat0, ur0, pp0),
                     (wr2t1, wr1t1, wat1, ur1, pp1)]):
                seed = sa[:, 2 * c + k:2 * c + k + 1]
                urb = (seed * wr2t) * _sig(urk)
                rb = rb + jnp.dot(urb, wr1t, preferred_element_type=f32)
                pbk = pb[:, 3 * k:3 * k + 3] * _sig(ppk[:, :3])
                pbk = jnp.concatenate(
                    [pbk, jnp.zeros_like(ppk[:, :5])], axis=1)
                rb = rb + jnp.dot(pbk, wat, preferred_element_type=f32)
            return rb

        sa_cols = jnp.concatenate([seeds_at, jnp.zeros((NP, 4), f32)], axis=1)
        rbar = _tc_map(
            rbar_fn,
            [pb0[0], pb0[1], pb1[0], pb1[1], ur_l[0], ur_l[1],
             p_l[0], p_l[1], sa_cols],
            [Wr2T[0], Wr1T[0], WaT8[0], Wr2T[1], Wr1T[1], WaT8[1]],
            [D], N_BLK)

        gbar = None
        for i in reversed(range(CONVS)):
            def aggb_fn(rbb, ub, w2t, w1t):
                vb = jnp.dot(rbb, w2t, preferred_element_type=f32)
                return jnp.dot(vb * _sig(ub), w1t, preferred_element_type=f32)

            aggbar = _tc_map(aggb_fn, [rbar, u_l[i]], [Wo2T[i], Wo1T[i]],
                             [D], N_BLK)
            Wbar, hbp = _sc_conv_bwd()(aggbar, h_l[i], Ws_l[i], a0, a1,
                                       zrow128)

            def rbup_fn(rbb, hb0, hb1, wt):
                return rbb + jnp.dot(hb0 + hb1, wt, preferred_element_type=f32)

            rbar = _tc_map(rbup_fn, [rbar, hbp[0], hbp[1]], [WinT[i]],
                           [D], N_BLK)

            def gb_fn(wbb, gvb, w2t, w1, b1, w1t, *gprev):
                sb = jnp.dot(wbb, w2t, preferred_element_type=f32)
                tb = sb * _sig(jnp.dot(gvb, w1, preferred_element_type=f32) + b1)
                out = jnp.dot(tb, w1t, preferred_element_type=f32)
                return out + gprev[0] if gprev else out

            gins = [Wbar, gvec] + ([gbar] if gbar is not None else [])

            def gb_wrap(wbb, gvb, *rest):
                auxs = rest[-4:]
                gp = rest[:-4]
                return gb_fn(wbb, gvb, auxs[0], auxs[1], auxs[2], auxs[3], *gp)

            gbar = _tc_map(gb_wrap, gins,
                           [Wg2T[i], Wg1[i], b2(bg1[i]), Wg1T[i]], [G], E_BLK)

        def delta_fn(dbe, gbb, gvb, db, x0b, x1b):
            d1 = db[:, 0:1]
            dbar = dbe + jnp.sum(
                gbb * gvb * (_mu_row() - d1), axis=1, keepdims=True) * (
                    1.0 / (sigma * sigma))
            coef = dbar / d1
            delta = coef * (x0b - x1b)
            return delta, -delta

        delta, ndelta = _tc_map(delta_fn, [dbar_eb, gbar, gvec, dcol, x0, x1],
                                [], [16, 16], E_BLK)
        gp0 = _sc_scatter_add(E, 16)(delta, a0, zrow16)
        gp1 = _sc_scatter_add(E, 16)(ndelta, a1, zrow16)
        gsum = _tc_map(lambda a, bq, cq, dq: a + bq + cq + dq,
                       [gp0[0], gp0[1], gp1[0], gp1[1]], [], [16], N_BLK)
        return gsum[:N, :3]

    g0 = backward(0)
    g1 = backward(1)
    return (se0, se1, g0, g1)


# speedup vs baseline: 5.4244x; 1.0487x over previous
import functools

import jax
import jax.numpy as jnp
from jax import lax
from jax.experimental import pallas as pl
from jax.experimental.pallas import tpu as pltpu
from jax.experimental.pallas import tpu_sc as plsc

N = 10000
E = 320000
B = 100
D = 128
G = 32
CONVS = 3
CUTOFF = 5.0

NP = 10240
NW = 32
C = 80
E_BLK = 3200
N_BLK = 2048

_LOG2 = 0.6931471805599453


def _ssp(x):
    return jnp.maximum(x, 0.0) + jnp.log1p(jnp.exp(-jnp.abs(x))) - _LOG2


def _sp(x):
    return jnp.maximum(x, 0.0) + jnp.log1p(jnp.exp(-jnp.abs(x)))


def _sig(x):
    return 1.0 / (1.0 + jnp.exp(-x))


def _mesh():
    return plsc.VectorSubcoreMesh(core_axis_name="c", subcore_axis_name="s")


@functools.lru_cache(maxsize=None)
def _sc_gather(n_idx, n_tab, dw):
    per_w = n_idx // NW
    nchunk = per_w // C

    @functools.partial(
        pl.kernel,
        mesh=_mesh(),
        out_type=jax.ShapeDtypeStruct((n_idx, dw), jnp.float32),
        compiler_params=pltpu.CompilerParams(use_tc_tiling_on_sc=False),
        scratch_types=[
            pltpu.VMEM((C,), jnp.int32),
            pltpu.VMEM((C, dw), jnp.float32),
            pltpu.SemaphoreType.DMA,
        ],
    )
    def k(table_hbm, idx_hbm, out_hbm, idx_v, rows_v, sem):
        wid = lax.axis_index("s") * 2 + lax.axis_index("c")
        base = wid * per_w

        def body(j, carry):
            off = base + j * C
            pltpu.sync_copy(idx_hbm.at[pl.ds(off, C)], idx_v)
            pltpu.async_copy(table_hbm.at[idx_v], rows_v, sem).wait()
            pltpu.sync_copy(rows_v, out_hbm.at[pl.ds(off, C)])
            return carry

        lax.fori_loop(0, nchunk, body, 0)

    return k


@functools.lru_cache(maxsize=None)
def _sc_scatter_add(n_vals, dw):
    per_w = n_vals // NW
    nchunk = per_w // C
    stripe = NP // 16

    @functools.partial(
        pl.kernel,
        mesh=_mesh(),
        out_type=jax.ShapeDtypeStruct((2, NP, dw), jnp.float32),
        compiler_params=pltpu.CompilerParams(use_tc_tiling_on_sc=False),
        scratch_types=[
            pltpu.VMEM((C,), jnp.int32),
            pltpu.VMEM((C, dw), jnp.float32),
            pltpu.VMEM_SHARED((NP, dw), jnp.float32),
            pltpu.SemaphoreType.DMA,
        ],
    )
    def k(vals_hbm, idx_hbm, zrow_hbm, out_hbm, idx_v, vals_v, acc_sh, sem):
        cid = lax.axis_index("c")
        sid = lax.axis_index("s")
        wid = sid * 2 + cid
        base = wid * per_w

        pltpu.sync_copy(zrow_hbm, vals_v)

        def zbody(j, carry):
            pltpu.sync_copy(vals_v, acc_sh.at[pl.ds(sid * stripe + j * C, C)])
            return carry

        lax.fori_loop(0, stripe // C, zbody, 0)
        plsc.subcore_barrier()

        def body(j, carry):
            off = base + j * C
            pltpu.sync_copy(idx_hbm.at[pl.ds(off, C)], idx_v)
            pltpu.sync_copy(vals_hbm.at[pl.ds(off, C)], vals_v)
            pltpu.sync_copy(vals_v, acc_sh.at[idx_v], add=True)
            return carry

        lax.fori_loop(0, nchunk, body, 0)
        plsc.subcore_barrier()
        pltpu.sync_copy(
            acc_sh.at[pl.ds(sid * stripe, stripe)],
            out_hbm.at[cid, pl.ds(sid * stripe, stripe)],
        )

    return k


CF = 40
_NCHF = (E // NW) // CF


@functools.lru_cache(maxsize=None)
def _sc_conv_fwd():
    per_w = E // NW
    nchunk = _NCHF
    stripe = NP // 16

    @functools.partial(
        pl.kernel,
        mesh=_mesh(),
        out_type=jax.ShapeDtypeStruct((2, NP, D), jnp.float32),
        compiler_params=pltpu.CompilerParams(use_tc_tiling_on_sc=False),
        scratch_types=[
            pltpu.VMEM((nchunk, CF), jnp.int32),
            pltpu.VMEM((nchunk, CF), jnp.int32),
            pltpu.VMEM((2, CF, D), jnp.float32),
            pltpu.VMEM((2, CF, D), jnp.float32),
            pltpu.VMEM_SHARED((NP, D), jnp.float32),
            pltpu.SemaphoreType.DMA,
            pltpu.SemaphoreType.DMA,
            pltpu.SemaphoreType.DMA,
            pltpu.SemaphoreType.DMA,
        ],
    )
    def k(h_hbm, we_hbm, a0r_hbm, a1r_hbm, zrow_hbm, out_hbm,
          idx0_v, idx1_v, hrow_v, werow_v, acc_sh, g0s, g1s, w0s, w1s):
        cid = lax.axis_index("c")
        sid = lax.axis_index("s")
        wid = sid * 2 + cid
        base = wid * per_w

        pltpu.sync_copy(zrow_hbm, hrow_v.at[0])

        def zbody(j, carry):
            pltpu.sync_copy(hrow_v.at[0],
                            acc_sh.at[pl.ds(sid * stripe + j * CF, CF)])
            return carry

        lax.fori_loop(0, stripe // CF, zbody, 0)
        pltpu.sync_copy(a0r_hbm.at[pl.ds(wid * nchunk, nchunk)], idx0_v)
        pltpu.sync_copy(a1r_hbm.at[pl.ds(wid * nchunk, nchunk)], idx1_v)
        plsc.subcore_barrier()

        def start(j, b, gs, ws):
            cg = pltpu.async_copy(h_hbm.at[idx1_v.at[j]], hrow_v.at[b], gs)
            cw = pltpu.async_copy(we_hbm.at[pl.ds(base + j * CF, CF)],
                                  werow_v.at[b], ws)
            return cg, cw

        def compute(j, b, cg, cw):
            cg.wait()
            cw.wait()

            def mul(i, carry2):
                for v in range(D // 16):
                    s = pl.ds(v * 16, 16)
                    hrow_v[b, i, s] = hrow_v[b, i, s] * werow_v[b, i, s]
                return carry2

            lax.fori_loop(0, CF, mul, 0)
            pltpu.sync_copy(hrow_v.at[b], acc_sh.at[idx0_v.at[j]], add=True)

        def body(t, carry):
            j0 = 2 * t
            j1 = j0 + 1
            cg0, cw0 = start(j0, 0, g0s, w0s)
            cg1, cw1 = start(j1, 1, g1s, w1s)
            compute(j0, 0, cg0, cw0)
            compute(j1, 1, cg1, cw1)
            return carry

        lax.fori_loop(0, nchunk // 2, body, 0)
        plsc.subcore_barrier()
        pltpu.sync_copy(
            acc_sh.at[pl.ds(sid * stripe, stripe)],
            out_hbm.at[cid, pl.ds(sid * stripe, stripe)],
        )

    return k


@functools.lru_cache(maxsize=None)
def _sc_conv_bwd():
    per_w = E // NW
    nchunk = _NCHF
    stripe = NP // 16

    @functools.partial(
        pl.kernel,
        mesh=_mesh(),
        out_type=(jax.ShapeDtypeStruct((E, D), jnp.float32),
                  jax.ShapeDtypeStruct((2, NP, D), jnp.float32)),
        compiler_params=pltpu.CompilerParams(use_tc_tiling_on_sc=False),
        scratch_types=[
            pltpu.VMEM((nchunk, CF), jnp.int32),
            pltpu.VMEM((nchunk, CF), jnp.int32),
            pltpu.VMEM((2, CF, D), jnp.float32),
            pltpu.VMEM((2, CF, D), jnp.float32),
            pltpu.VMEM((CF, D), jnp.float32),
            pltpu.VMEM_SHARED((NP, D), jnp.float32),
            pltpu.SemaphoreType.DMA,
            pltpu.SemaphoreType.DMA,
            pltpu.SemaphoreType.DMA,
            pltpu.SemaphoreType.DMA,
        ],
    )
    def k(ab_hbm, h_hbm, we_hbm, a0r_hbm, a1r_hbm, zrow_hbm,
          wbar_hbm, out_hbm, idx0_v, idx1_v, mbrow_v, hrow_v, werow_v,
          acc_sh, m0s, m1s, g0s, g1s):
        cid = lax.axis_index("c")
        sid = lax.axis_index("s")
        wid = sid * 2 + cid
        base = wid * per_w

        pltpu.sync_copy(zrow_hbm, werow_v)

        def zbody(j, carry):
            pltpu.sync_copy(werow_v,
                            acc_sh.at[pl.ds(sid * stripe + j * CF, CF)])
            return carry

        lax.fori_loop(0, stripe // CF, zbody, 0)
        pltpu.sync_copy(a0r_hbm.at[pl.ds(wid * nchunk, nchunk)], idx0_v)
        pltpu.sync_copy(a1r_hbm.at[pl.ds(wid * nchunk, nchunk)], idx1_v)
        plsc.subcore_barrier()

        def start(j, b, ms, gs):
            cm = pltpu.async_copy(ab_hbm.at[idx0_v.at[j]], mbrow_v.at[b], ms)
            cg = pltpu.async_copy(h_hbm.at[idx1_v.at[j]], hrow_v.at[b], gs)
            return cm, cg

        def compute(j, b, cm, cg):
            pltpu.sync_copy(we_hbm.at[pl.ds(base + j * CF, CF)], werow_v)
            cm.wait()
            cg.wait()

            def mul(i, carry2):
                for v in range(D // 16):
                    s = pl.ds(v * 16, 16)
                    mb = mbrow_v[b, i, s]
                    hrow_v[b, i, s] = mb * hrow_v[b, i, s]
                    werow_v[i, s] = mb * werow_v[i, s]
                return carry2

            lax.fori_loop(0, CF, mul, 0)
            pltpu.sync_copy(hrow_v.at[b], wbar_hbm.at[pl.ds(base + j * CF, CF)])
            pltpu.sync_copy(werow_v, acc_sh.at[idx1_v.at[j]], add=True)

        def body(t, carry):
            j0 = 2 * t
            j1 = j0 + 1
            c0 = start(j0, 0, m0s, g0s)
            c1 = start(j1, 1, m1s, g1s)
            compute(j0, 0, *c0)
            compute(j1, 1, *c1)
            return carry

        lax.fori_loop(0, nchunk // 2, body, 0)
        plsc.subcore_barrier()
        pltpu.sync_copy(
            acc_sh.at[pl.ds(sid * stripe, stripe)],
            out_hbm.at[cid, pl.ds(sid * stripe, stripe)],
        )

    return k


def _tc_map(fn, rows, auxs, out_cols, blk):
    R = rows[0].shape[0]
    grid = R // blk

    def row_spec(x):
        nd = x.ndim
        bs = (blk,) + x.shape[1:]
        return pl.BlockSpec(bs, lambda i, nd=nd: (i,) + (0,) * (nd - 1))

    def aux_spec(a):
        nd = a.ndim
        return pl.BlockSpec(a.shape, lambda i, nd=nd: (0,) * nd)

    in_specs = [row_spec(x) for x in rows] + [aux_spec(a) for a in auxs]
    out_specs = [pl.BlockSpec((blk, c), lambda i: (i, 0)) for c in out_cols]
    out_shape = [jax.ShapeDtypeStruct((R, c), jnp.float32) for c in out_cols]
    n_in = len(rows) + len(auxs)

    def body(*refs):
        vals = fn(*[ref[...] for ref in refs[:n_in]])
        if not isinstance(vals, (tuple, list)):
            vals = (vals,)
        for oref, v in zip(refs[n_in:], vals):
            oref[...] = v

    res = pl.pallas_call(
        body, grid=(grid,), in_specs=in_specs, out_specs=out_specs,
        out_shape=out_shape,
    )(*rows, *auxs)
    return res if len(out_cols) > 1 else res[0]


def _tc_whole(fn, ins, out_shapes):
    in_specs = [pl.BlockSpec(x.shape, lambda *_, nd=x.ndim: (0,) * nd) for x in ins]
    out_specs = [pl.BlockSpec(s, lambda *_, nd=len(s): (0,) * nd) for s in out_shapes]
    out_shape = [jax.ShapeDtypeStruct(s, jnp.float32) for s in out_shapes]
    n_in = len(ins)

    def body(*refs):
        vals = fn(*[ref[...] for ref in refs[:n_in]])
        if not isinstance(vals, (tuple, list)):
            vals = (vals,)
        for oref, v in zip(refs[n_in:], vals):
            oref[...] = v

    res = pl.pallas_call(
        body, in_specs=in_specs, out_specs=out_specs, out_shape=out_shape,
    )(*ins)
    return res if len(out_shapes) > 1 else res[0]


def kernel(nxyz, nbr_list, num_atoms, embed, Wg1, bg1, Wg2, bg2, Win, bin,
           Wo1, bo1, Wo2, bo2, Wr1, br1, Wr2, br2, Wa, ba):
    f32 = jnp.float32
    sigma = float(CUTOFF / (G - 1))
    inv2s2 = 1.0 / (2.0 * sigma * sigma)

    def _mu_row():
        return lax.broadcasted_iota(jnp.int32, (1, G), 1).astype(f32) * (
            CUTOFF / (G - 1))

    z = nxyz[:, 0].astype(jnp.int32)
    xyz = nxyz[:, 1:4].astype(f32)
    a0 = nbr_list[:, 0].astype(jnp.int32)
    a1 = nbr_list[:, 1].astype(jnp.int32)

    zp = jnp.pad(z, (0, NP - N))
    xyzp = jnp.pad(xyz, ((0, NP - N), (0, 13)))
    a0r = a0.reshape(E // CF, CF)
    a1r = a1.reshape(E // CF, CF)
    zrowF = jnp.zeros((CF, D), f32)
    zrow16 = jnp.zeros((C, 16), f32)
    zrow128 = jnp.zeros((C, D), f32)

    r = _sc_gather(NP, 100, D)(embed, zp)
    x0 = _sc_gather(E, NP, 16)(xyzp, a0)
    x1 = _sc_gather(E, NP, 16)(xyzp, a1)

    def geom_fn(x0b, x1b):
        df = x0b - x1b
        d = jnp.sqrt(jnp.sum(df * df, axis=1, keepdims=True) + 1e-12)
        gv = jnp.exp(-((d - _mu_row()) ** 2) * inv2s2)
        return d, gv

    dcol, gvec = _tc_map(geom_fn, [x0, x1], [], [1, G], E_BLK)

    b2 = lambda v: v.reshape(1, -1)
    Ws_l, h_l, u_l = [], [], []
    for i in range(CONVS):
        def filt_fn(gb, W1, b1, W2, bb2):
            return _ssp(jnp.dot(gb, W1, preferred_element_type=f32) + b1) @ W2 + bb2

        We = _tc_map(filt_fn, [gvec], [Wg1[i], b2(bg1[i]), Wg2[i], b2(bg2[i])],
                     [D], E_BLK)

        def h_fn(rb, W, bb):
            return jnp.dot(rb, W, preferred_element_type=f32) + bb

        h = _tc_map(h_fn, [r], [Win[i], b2(bin[i])], [D], N_BLK)
        aggp = _sc_conv_fwd()(h, We, a0r, a1r, zrowF)

        def out_fn(rb, g0b, g1b, W1, b1, W2, bb2):
            u = jnp.dot(g0b + g1b, W1, preferred_element_type=f32) + b1
            rn = rb + jnp.dot(_ssp(u), W2, preferred_element_type=f32) + bb2
            return u, rn

        u, r = _tc_map(out_fn, [r, aggp[0], aggp[1]],
                       [Wo1[i], b2(bo1[i]), Wo2[i], b2(bo2[i])], [D, D], N_BLK)
        Ws_l.append(We)
        h_l.append(h)
        u_l.append(u)

    Wa8 = jnp.pad(Wa, ((0, 0), (0, 0), (0, 5)))
    ba8 = jnp.pad(ba, ((0, 0), (0, 5)))
    ur_l, p_l, ae_l, pa_l = [], [], [], []
    for k in range(2):
        def ro_fn(rb, W1, b1, W2, bb2, Wap, bap):
            ur = jnp.dot(rb, W1, preferred_element_type=f32) + b1
            ae = jnp.dot(_ssp(ur), W2, preferred_element_type=f32) + bb2
            p = jnp.dot(rb, Wap, preferred_element_type=f32) + bap
            return ur, ae, p, _sp(p)

        ur, ae, p, pa = _tc_map(
            ro_fn, [r],
            [Wr1[k], b2(br1[k]), Wr2[k], b2(br2[k]), Wa8[k], b2(ba8[k])],
            [64, 1, 8, 8], N_BLK)
        ur_l.append(ur); p_l.append(p); ae_l.append(ae); pa_l.append(pa)

    patom = jnp.concatenate(
        [pa_l[0][:, :3], pa_l[1][:, :3], jnp.zeros((NP, 10), f32)], axis=1)
    P0 = _sc_gather(E, NP, 16)(patom, a0)
    P1 = _sc_gather(E, NP, 16)(patom, a1)

    def eb_fn(p0b, p1b, db):
        cols = []
        d1 = db[:, 0]
        for k in range(2):
            Dp = 0.5 * (p0b[:, 3 * k] + p1b[:, 3 * k])
            ap = 0.5 * (p0b[:, 3 * k + 1] + p1b[:, 3 * k + 1])
            r0 = 0.5 * (p0b[:, 3 * k + 2] + p1b[:, 3 * k + 2])
            q = 1.0 - jnp.exp(-ap * (d1 - r0))
            cols.append((Dp * q * q)[:, None])
        return jnp.concatenate(cols + [jnp.zeros_like(p0b[:, :14])], axis=1)

    ebrow = _tc_map(eb_fn, [P0, P1, dcol], [], [16], E_BLK)
    ebp = _sc_scatter_add(E, 16)(ebrow, a0, zrow16)

    def mol_fn(ae0b, ae1b, e00, e01, e10, e11):
        m0 = jnp.sum(ae0b, axis=1) + jnp.sum(e00 + e01, axis=1)
        m1 = jnp.sum(ae1b, axis=1) + jnp.sum(e10 + e11, axis=1)
        se0 = jnp.minimum(m0, m1)
        se1 = jnp.maximum(m0, m1)
        t0 = (m0 <= m1).astype(f32)
        out = jnp.stack([se0, se1, t0, 1.0 - t0], axis=1)
        return jnp.pad(out, ((0, 0), (0, 4)))

    mol_ins = [ae_l[0][:N, 0].reshape(B, N // B), ae_l[1][:N, 0].reshape(B, N // B),
               ebp[0, :N, 0].reshape(B, N // B), ebp[1, :N, 0].reshape(B, N // B),
               ebp[0, :N, 1].reshape(B, N // B), ebp[1, :N, 1].reshape(B, N // B)]
    mol = _tc_whole(mol_fn, mol_ins, [(B, 8)])
    se0, se1 = mol[:, 0], mol[:, 1]
    w = mol[:, 2:4]
    seeds_at = jnp.repeat(
        jnp.concatenate([w, 1.0 - w], axis=1), N // B, axis=0)
    seeds_at = jnp.pad(seeds_at, ((0, NP - N), (0, 0)))

    patom2 = jnp.concatenate(
        [pa_l[0][:, :3], pa_l[1][:, :3], seeds_at, jnp.zeros((NP, 6), f32)],
        axis=1)
    P0b = _sc_gather(E, NP, 16)(patom2, a0)
    P1b = _sc_gather(E, NP, 16)(patom2, a1)

    Wr2T = [b2(Wr2[k][:, 0]) for k in range(2)]
    Wr1T = [Wr1[k].T for k in range(2)]
    WaT8 = [jnp.pad(Wa[k].T, ((0, 5), (0, 0))) for k in range(2)]
    Wo2T = [Wo2[i].T for i in range(CONVS)]
    Wo1T = [Wo1[i].T for i in range(CONVS)]
    WinT = [Win[i].T for i in range(CONVS)]
    Wg2T = [Wg2[i].T for i in range(CONVS)]
    Wg1T = [Wg1[i].T for i in range(CONVS)]

    def backward(c):
        def ebb_fn(p0b, p1b, db):
            d1 = db[:, 0]
            dbar = jnp.zeros_like(d1)
            cols = []
            for k in range(2):
                seed = p0b[:, 6 + 2 * c + k]
                Dp = 0.5 * (p0b[:, 3 * k] + p1b[:, 3 * k])
                ap = 0.5 * (p0b[:, 3 * k + 1] + p1b[:, 3 * k + 1])
                r0 = 0.5 * (p0b[:, 3 * k + 2] + p1b[:, 3 * k + 2])
                ex = jnp.exp(-ap * (d1 - r0))
                q = 1.0 - ex
                Dpb = seed * q * q
                qb = seed * Dp * 2.0 * q
                apb = qb * ex * (d1 - r0)
                dbar = dbar + qb * ex * ap
                r0b = -qb * ex * ap
                cols += [(0.5 * Dpb)[:, None], (0.5 * apb)[:, None],
                         (0.5 * r0b)[:, None]]
            crow = jnp.concatenate(cols + [jnp.zeros_like(p0b[:, :10])], axis=1)
            return dbar[:, None], crow

        dbar_eb, crow = _tc_map(ebb_fn, [P0b, P1b, dcol], [], [1, 16], E_BLK)
        pb0 = _sc_scatter_add(E, 16)(crow, a0, zrow16)
        pb1 = _sc_scatter_add(E, 16)(crow, a1, zrow16)

        def rbar_fn(pba, pbb, pbc, pbd, ur0, ur1, pp0, pp1, sa,
                    wr2t0, wr1t0, wat0, wr2t1, wr1t1, wat1):
            pb = pba + pbb + pbc + pbd
            rb = jnp.zeros((pba.shape[0], D), f32)
            for k, (wr2t, wr1t, wat, urk, ppk) in enumerate(
                    [(wr2t0, wr1t0, wat0, ur0, pp0),
                     (wr2t1, wr1t1, wat1, ur1, pp1)]):
                seed = sa[:, 2 * c + k:2 * c + k + 1]
                urb = (seed * wr2t) * _sig(urk)
                rb = rb + jnp.dot(urb, wr1t, preferred_element_type=f32)
                pbk = pb[:, 3 * k:3 * k + 3] * _sig(ppk[:, :3])
                pbk = jnp.concatenate(
                    [pbk, jnp.zeros_like(ppk[:, :5])], axis=1)
                rb = rb + jnp.dot(pbk, wat, preferred_element_type=f32)
            return rb

        sa_cols = jnp.concatenate([seeds_at, jnp.zeros((NP, 4), f32)], axis=1)
        rbar = _tc_map(
            rbar_fn,
            [pb0[0], pb0[1], pb1[0], pb1[1], ur_l[0], ur_l[1],
             p_l[0], p_l[1], sa_cols],
            [Wr2T[0], Wr1T[0], WaT8[0], Wr2T[1], Wr1T[1], WaT8[1]],
            [D], N_BLK)

        gbar = None
        for i in reversed(range(CONVS)):
            def aggb_fn(rbb, ub, w2t, w1t):
                vb = jnp.dot(rbb, w2t, preferred_element_type=f32)
                return jnp.dot(vb * _sig(ub), w1t, preferred_element_type=f32)

            aggbar = _tc_map(aggb_fn, [rbar, u_l[i]], [Wo2T[i], Wo1T[i]],
                             [D], N_BLK)
            Wbar, hbp = _sc_conv_bwd()(aggbar, h_l[i], Ws_l[i], a0r, a1r,
                                       zrowF)

            def rbup_fn(rbb, hb0, hb1, wt):
                return rbb + jnp.dot(hb0 + hb1, wt, preferred_element_type=f32)

            rbar = _tc_map(rbup_fn, [rbar, hbp[0], hbp[1]], [WinT[i]],
                           [D], N_BLK)

            def gb_fn(wbb, gvb, w2t, w1, b1, w1t, *gprev):
                sb = jnp.dot(wbb, w2t, preferred_element_type=f32)
                tb = sb * _sig(jnp.dot(gvb, w1, preferred_element_type=f32) + b1)
                out = jnp.dot(tb, w1t, preferred_element_type=f32)
                return out + gprev[0] if gprev else out

            gins = [Wbar, gvec] + ([gbar] if gbar is not None else [])

            def gb_wrap(wbb, gvb, *rest):
                auxs = rest[-4:]
                gp = rest[:-4]
                return gb_fn(wbb, gvb, auxs[0], auxs[1], auxs[2], auxs[3], *gp)

            gbar = _tc_map(gb_wrap, gins,
                           [Wg2T[i], Wg1[i], b2(bg1[i]), Wg1T[i]], [G], E_BLK)

        def delta_fn(dbe, gbb, gvb, db, x0b, x1b):
            d1 = db[:, 0:1]
            dbar = dbe + jnp.sum(
                gbb * gvb * (_mu_row() - d1), axis=1, keepdims=True) * (
                    1.0 / (sigma * sigma))
            coef = dbar / d1
            delta = coef * (x0b - x1b)
            return delta, -delta

        delta, ndelta = _tc_map(delta_fn, [dbar_eb, gbar, gvec, dcol, x0, x1],
                                [], [16, 16], E_BLK)
        gp0 = _sc_scatter_add(E, 16)(delta, a0, zrow16)
        gp1 = _sc_scatter_add(E, 16)(ndelta, a1, zrow16)
        gsum = _tc_map(lambda a, bq, cq, dq: a + bq + cq + dq,
                       [gp0[0], gp0[1], gp1[0], gp1[1]], [], [16], N_BLK)
        return gsum[:N, :3]

    g0 = backward(0)
    g1 = backward(1)
    return (se0, se1, g0, g1)


# speedup vs baseline: 6.0750x; 1.1199x over previous
import functools

import jax
import jax.numpy as jnp
from jax import lax
from jax.experimental import pallas as pl
from jax.experimental.pallas import tpu as pltpu
from jax.experimental.pallas import tpu_sc as plsc

N = 10000
E = 320000
B = 100
D = 128
G = 32
CONVS = 3
CUTOFF = 5.0

NP = 10240
NW = 32
C = 80
E_BLK = 3200
N_BLK = 2048

_LOG2 = 0.6931471805599453


def _ssp(x):
    return jnp.maximum(x, 0.0) + jnp.log1p(jnp.exp(-jnp.abs(x))) - _LOG2


def _sp(x):
    return jnp.maximum(x, 0.0) + jnp.log1p(jnp.exp(-jnp.abs(x)))


def _sig(x):
    return 1.0 / (1.0 + jnp.exp(-x))


def _mesh():
    return plsc.VectorSubcoreMesh(core_axis_name="c", subcore_axis_name="s")


@functools.lru_cache(maxsize=None)
def _sc_gather(n_idx, n_tab, dw):
    per_w = n_idx // NW
    nchunk = per_w // C

    @functools.partial(
        pl.kernel,
        mesh=_mesh(),
        out_type=jax.ShapeDtypeStruct((n_idx, dw), jnp.float32),
        compiler_params=pltpu.CompilerParams(use_tc_tiling_on_sc=False),
        scratch_types=[
            pltpu.VMEM((C,), jnp.int32),
            pltpu.VMEM((C, dw), jnp.float32),
            pltpu.SemaphoreType.DMA,
        ],
    )
    def k(table_hbm, idx_hbm, out_hbm, idx_v, rows_v, sem):
        wid = lax.axis_index("s") * 2 + lax.axis_index("c")
        base = wid * per_w

        def body(j, carry):
            off = base + j * C
            pltpu.sync_copy(idx_hbm.at[pl.ds(off, C)], idx_v)
            pltpu.async_copy(table_hbm.at[idx_v], rows_v, sem).wait()
            pltpu.sync_copy(rows_v, out_hbm.at[pl.ds(off, C)])
            return carry

        lax.fori_loop(0, nchunk, body, 0)

    return k


@functools.lru_cache(maxsize=None)
def _sc_scatter_add(n_vals, dw):
    per_w = n_vals // NW
    nchunk = per_w // C
    stripe = NP // 16

    @functools.partial(
        pl.kernel,
        mesh=_mesh(),
        out_type=jax.ShapeDtypeStruct((2, NP, dw), jnp.float32),
        compiler_params=pltpu.CompilerParams(use_tc_tiling_on_sc=False),
        scratch_types=[
            pltpu.VMEM((C,), jnp.int32),
            pltpu.VMEM((C, dw), jnp.float32),
            pltpu.VMEM_SHARED((NP, dw), jnp.float32),
            pltpu.SemaphoreType.DMA,
        ],
    )
    def k(vals_hbm, idx_hbm, zrow_hbm, out_hbm, idx_v, vals_v, acc_sh, sem):
        cid = lax.axis_index("c")
        sid = lax.axis_index("s")
        wid = sid * 2 + cid
        base = wid * per_w

        pltpu.sync_copy(zrow_hbm, vals_v)

        def zbody(j, carry):
            pltpu.sync_copy(vals_v, acc_sh.at[pl.ds(sid * stripe + j * C, C)])
            return carry

        lax.fori_loop(0, stripe // C, zbody, 0)
        plsc.subcore_barrier()

        def body(j, carry):
            off = base + j * C
            pltpu.sync_copy(idx_hbm.at[pl.ds(off, C)], idx_v)
            pltpu.sync_copy(vals_hbm.at[pl.ds(off, C)], vals_v)
            pltpu.sync_copy(vals_v, acc_sh.at[idx_v], add=True)
            return carry

        lax.fori_loop(0, nchunk, body, 0)
        plsc.subcore_barrier()
        pltpu.sync_copy(
            acc_sh.at[pl.ds(sid * stripe, stripe)],
            out_hbm.at[cid, pl.ds(sid * stripe, stripe)],
        )

    return k


CF = 40
_NCHF = (E // NW) // CF


@functools.lru_cache(maxsize=None)
def _sc_pair_gather(dw):
    per_w = E // NW
    nchunk = per_w // C

    @functools.partial(
        pl.kernel,
        mesh=_mesh(),
        out_type=(jax.ShapeDtypeStruct((E, dw), jnp.float32),
                  jax.ShapeDtypeStruct((E, dw), jnp.float32)),
        compiler_params=pltpu.CompilerParams(use_tc_tiling_on_sc=False),
        scratch_types=[
            pltpu.VMEM((nchunk, C), jnp.int32),
            pltpu.VMEM((nchunk, C), jnp.int32),
            pltpu.VMEM((2, C, dw), jnp.float32),
            pltpu.VMEM((2, C, dw), jnp.float32),
            pltpu.SemaphoreType.DMA,
            pltpu.SemaphoreType.DMA,
            pltpu.SemaphoreType.DMA,
            pltpu.SemaphoreType.DMA,
            pltpu.SemaphoreType.DMA,
            pltpu.SemaphoreType.DMA,
            pltpu.SemaphoreType.DMA,
            pltpu.SemaphoreType.DMA,
        ],
    )
    def k(tab_hbm, a0r_hbm, a1r_hbm, o0_hbm, o1_hbm,
          idx0_v, idx1_v, r0_v, r1_v,
          g00, g01, g10, g11, w00, w01, w10, w11):
        cid = lax.axis_index("c")
        sid = lax.axis_index("s")
        wid = sid * 2 + cid
        base = wid * per_w
        pltpu.sync_copy(a0r_hbm.at[pl.ds(wid * nchunk, nchunk)], idx0_v)
        pltpu.sync_copy(a1r_hbm.at[pl.ds(wid * nchunk, nchunk)], idx1_v)

        def start(j, b, ga, gb):
            c0 = pltpu.async_copy(tab_hbm.at[idx0_v.at[j]], r0_v.at[b], ga)
            c1 = pltpu.async_copy(tab_hbm.at[idx1_v.at[j]], r1_v.at[b], gb)
            return c0, c1

        def drain(j, b, c0, c1, wa, wb):
            c0.wait()
            c1.wait()
            s = pl.ds(base + j * C, C)
            w0 = pltpu.async_copy(r0_v.at[b], o0_hbm.at[s], wa)
            w1 = pltpu.async_copy(r1_v.at[b], o1_hbm.at[s], wb)
            return w0, w1

        def body(t, carry):
            j0 = 2 * t
            j1 = j0 + 1
            ca = start(j0, 0, g00, g01)
            cb = start(j1, 1, g10, g11)
            wa = drain(j0, 0, *ca, w00, w01)
            wb = drain(j1, 1, *cb, w10, w11)
            for w in (*wa, *wb):
                w.wait()
            return carry

        lax.fori_loop(0, nchunk // 2, body, 0)
        ca = start(nchunk - 1, 0, g00, g01)
        wa = drain(nchunk - 1, 0, *ca, w00, w01)
        for w in wa:
            w.wait()

    return k


@functools.lru_cache(maxsize=None)
def _sc_scat16(two_idx, two_acc):
    per_w = E // NW
    nchunk = per_w // C
    stripe = NP // 16
    dw = 16
    nacc = 2 if two_acc else 1
    oshape = (2, 2, NP, dw) if two_acc else (2, NP, dw)

    scratch = [
        pltpu.VMEM((nchunk, C), jnp.int32),
        pltpu.VMEM((nchunk, C), jnp.int32),
        pltpu.VMEM((2, C, dw), jnp.float32),
    ] + [pltpu.VMEM_SHARED((NP, dw), jnp.float32) for _ in range(nacc)] + [
        pltpu.SemaphoreType.DMA,
        pltpu.SemaphoreType.DMA,
    ]

    @functools.partial(
        pl.kernel,
        mesh=_mesh(),
        out_type=jax.ShapeDtypeStruct(oshape, jnp.float32),
        compiler_params=pltpu.CompilerParams(use_tc_tiling_on_sc=False),
        scratch_types=scratch,
    )
    def k(vals_hbm, a0r_hbm, a1r_hbm, zrow_hbm, out_hbm,
          idx0_v, idx1_v, vals_v, *rest):
        accs = rest[:nacc]
        s0, s1 = rest[nacc], rest[nacc + 1]
        cid = lax.axis_index("c")
        sid = lax.axis_index("s")
        wid = sid * 2 + cid
        base = wid * per_w

        pltpu.sync_copy(zrow_hbm, vals_v.at[0])

        def zbody(j, carry):
            for a in range(nacc):
                pltpu.sync_copy(
                    vals_v.at[0],
                    accs[a].at[pl.ds(sid * stripe + j * C, C)])
            return carry

        lax.fori_loop(0, stripe // C, zbody, 0)
        pltpu.sync_copy(a0r_hbm.at[pl.ds(wid * nchunk, nchunk)], idx0_v)
        if two_idx:
            pltpu.sync_copy(a1r_hbm.at[pl.ds(wid * nchunk, nchunk)], idx1_v)
        plsc.subcore_barrier()

        def start(j, b, sem):
            return pltpu.async_copy(
                vals_hbm.at[pl.ds(base + j * C, C)], vals_v.at[b], sem)

        def drain(j, b, cp):
            cp.wait()
            pltpu.sync_copy(vals_v.at[b], accs[0].at[idx0_v.at[j]], add=True)
            if two_idx:
                pltpu.sync_copy(vals_v.at[b], accs[nacc - 1].at[idx1_v.at[j]],
                                add=True)

        def body(t, carry):
            j0 = 2 * t
            j1 = j0 + 1
            c0 = start(j0, 0, s0)
            c1 = start(j1, 1, s1)
            drain(j0, 0, c0)
            drain(j1, 1, c1)
            return carry

        lax.fori_loop(0, nchunk // 2, body, 0)
        c0 = start(nchunk - 1, 0, s0)
        drain(nchunk - 1, 0, c0)
        plsc.subcore_barrier()
        for a in range(nacc):
            src = accs[a].at[pl.ds(sid * stripe, stripe)]
            if two_acc:
                pltpu.sync_copy(src, out_hbm.at[cid, a,
                                                pl.ds(sid * stripe, stripe)])
            else:
                pltpu.sync_copy(src, out_hbm.at[cid,
                                                pl.ds(sid * stripe, stripe)])

    return k


@functools.lru_cache(maxsize=None)
def _sc_conv_fwd():
    per_w = E // NW
    nchunk = _NCHF
    stripe = NP // 16

    @functools.partial(
        pl.kernel,
        mesh=_mesh(),
        out_type=jax.ShapeDtypeStruct((2, NP, D), jnp.float32),
        compiler_params=pltpu.CompilerParams(use_tc_tiling_on_sc=False),
        scratch_types=[
            pltpu.VMEM((nchunk, CF), jnp.int32),
            pltpu.VMEM((nchunk, CF), jnp.int32),
            pltpu.VMEM((2, CF, D), jnp.float32),
            pltpu.VMEM((2, CF, D), jnp.float32),
            pltpu.VMEM_SHARED((NP, D), jnp.float32),
            pltpu.SemaphoreType.DMA,
            pltpu.SemaphoreType.DMA,
            pltpu.SemaphoreType.DMA,
            pltpu.SemaphoreType.DMA,
        ],
    )
    def k(h_hbm, we_hbm, a0r_hbm, a1r_hbm, zrow_hbm, out_hbm,
          idx0_v, idx1_v, hrow_v, werow_v, acc_sh, g0s, g1s, w0s, w1s):
        cid = lax.axis_index("c")
        sid = lax.axis_index("s")
        wid = sid * 2 + cid
        base = wid * per_w

        pltpu.sync_copy(zrow_hbm, hrow_v.at[0])

        def zbody(j, carry):
            pltpu.sync_copy(hrow_v.at[0],
                            acc_sh.at[pl.ds(sid * stripe + j * CF, CF)])
            return carry

        lax.fori_loop(0, stripe // CF, zbody, 0)
        pltpu.sync_copy(a0r_hbm.at[pl.ds(wid * nchunk, nchunk)], idx0_v)
        pltpu.sync_copy(a1r_hbm.at[pl.ds(wid * nchunk, nchunk)], idx1_v)
        plsc.subcore_barrier()

        def start(j, b, gs, ws):
            cg = pltpu.async_copy(h_hbm.at[idx1_v.at[j]], hrow_v.at[b], gs)
            cw = pltpu.async_copy(we_hbm.at[pl.ds(base + j * CF, CF)],
                                  werow_v.at[b], ws)
            return cg, cw

        def compute(j, b, cg, cw):
            cg.wait()
            cw.wait()

            def mul(i, carry2):
                for v in range(D // 16):
                    s = pl.ds(v * 16, 16)
                    hrow_v[b, i, s] = hrow_v[b, i, s] * werow_v[b, i, s]
                return carry2

            lax.fori_loop(0, CF, mul, 0)
            pltpu.sync_copy(hrow_v.at[b], acc_sh.at[idx0_v.at[j]], add=True)

        def body(t, carry):
            j0 = 2 * t
            j1 = j0 + 1
            cg0, cw0 = start(j0, 0, g0s, w0s)
            cg1, cw1 = start(j1, 1, g1s, w1s)
            compute(j0, 0, cg0, cw0)
            compute(j1, 1, cg1, cw1)
            return carry

        lax.fori_loop(0, nchunk // 2, body, 0)
        plsc.subcore_barrier()
        pltpu.sync_copy(
            acc_sh.at[pl.ds(sid * stripe, stripe)],
            out_hbm.at[cid, pl.ds(sid * stripe, stripe)],
        )

    return k


@functools.lru_cache(maxsize=None)
def _sc_conv_bwd():
    per_w = E // NW
    nchunk = _NCHF
    stripe = NP // 16

    @functools.partial(
        pl.kernel,
        mesh=_mesh(),
        out_type=(jax.ShapeDtypeStruct((E, D), jnp.float32),
                  jax.ShapeDtypeStruct((2, NP, D), jnp.float32)),
        compiler_params=pltpu.CompilerParams(use_tc_tiling_on_sc=False),
        scratch_types=[
            pltpu.VMEM((nchunk, CF), jnp.int32),
            pltpu.VMEM((nchunk, CF), jnp.int32),
            pltpu.VMEM((2, CF, D), jnp.float32),
            pltpu.VMEM((2, CF, D), jnp.float32),
            pltpu.VMEM((CF, D), jnp.float32),
            pltpu.VMEM_SHARED((NP, D), jnp.float32),
            pltpu.SemaphoreType.DMA,
            pltpu.SemaphoreType.DMA,
            pltpu.SemaphoreType.DMA,
            pltpu.SemaphoreType.DMA,
        ],
    )
    def k(ab_hbm, h_hbm, we_hbm, a0r_hbm, a1r_hbm, zrow_hbm,
          wbar_hbm, out_hbm, idx0_v, idx1_v, mbrow_v, hrow_v, werow_v,
          acc_sh, m0s, m1s, g0s, g1s):
        cid = lax.axis_index("c")
        sid = lax.axis_index("s")
        wid = sid * 2 + cid
        base = wid * per_w

        pltpu.sync_copy(zrow_hbm, werow_v)

        def zbody(j, carry):
            pltpu.sync_copy(werow_v,
                            acc_sh.at[pl.ds(sid * stripe + j * CF, CF)])
            return carry

        lax.fori_loop(0, stripe // CF, zbody, 0)
        pltpu.sync_copy(a0r_hbm.at[pl.ds(wid * nchunk, nchunk)], idx0_v)
        pltpu.sync_copy(a1r_hbm.at[pl.ds(wid * nchunk, nchunk)], idx1_v)
        plsc.subcore_barrier()

        def start(j, b, ms, gs):
            cm = pltpu.async_copy(ab_hbm.at[idx0_v.at[j]], mbrow_v.at[b], ms)
            cg = pltpu.async_copy(h_hbm.at[idx1_v.at[j]], hrow_v.at[b], gs)
            return cm, cg

        def compute(j, b, cm, cg):
            pltpu.sync_copy(we_hbm.at[pl.ds(base + j * CF, CF)], werow_v)
            cm.wait()
            cg.wait()

            def mul(i, carry2):
                for v in range(D // 16):
                    s = pl.ds(v * 16, 16)
                    mb = mbrow_v[b, i, s]
                    hrow_v[b, i, s] = mb * hrow_v[b, i, s]
                    werow_v[i, s] = mb * werow_v[i, s]
                return carry2

            lax.fori_loop(0, CF, mul, 0)
            pltpu.sync_copy(hrow_v.at[b], wbar_hbm.at[pl.ds(base + j * CF, CF)])
            pltpu.sync_copy(werow_v, acc_sh.at[idx1_v.at[j]], add=True)

        def body(t, carry):
            j0 = 2 * t
            j1 = j0 + 1
            c0 = start(j0, 0, m0s, g0s)
            c1 = start(j1, 1, m1s, g1s)
            compute(j0, 0, *c0)
            compute(j1, 1, *c1)
            return carry

        lax.fori_loop(0, nchunk // 2, body, 0)
        plsc.subcore_barrier()
        pltpu.sync_copy(
            acc_sh.at[pl.ds(sid * stripe, stripe)],
            out_hbm.at[cid, pl.ds(sid * stripe, stripe)],
        )

    return k


def _tc_map(fn, rows, auxs, out_cols, blk):
    R = rows[0].shape[0]
    grid = R // blk

    def row_spec(x):
        nd = x.ndim
        bs = (blk,) + x.shape[1:]
        return pl.BlockSpec(bs, lambda i, nd=nd: (i,) + (0,) * (nd - 1))

    def aux_spec(a):
        nd = a.ndim
        return pl.BlockSpec(a.shape, lambda i, nd=nd: (0,) * nd)

    in_specs = [row_spec(x) for x in rows] + [aux_spec(a) for a in auxs]
    out_specs = [pl.BlockSpec((blk, c), lambda i: (i, 0)) for c in out_cols]
    out_shape = [jax.ShapeDtypeStruct((R, c), jnp.float32) for c in out_cols]
    n_in = len(rows) + len(auxs)

    def body(*refs):
        vals = fn(*[ref[...] for ref in refs[:n_in]])
        if not isinstance(vals, (tuple, list)):
            vals = (vals,)
        for oref, v in zip(refs[n_in:], vals):
            oref[...] = v

    res = pl.pallas_call(
        body, grid=(grid,), in_specs=in_specs, out_specs=out_specs,
        out_shape=out_shape,
    )(*rows, *auxs)
    return res if len(out_cols) > 1 else res[0]


def _tc_whole(fn, ins, out_shapes):
    in_specs = [pl.BlockSpec(x.shape, lambda *_, nd=x.ndim: (0,) * nd) for x in ins]
    out_specs = [pl.BlockSpec(s, lambda *_, nd=len(s): (0,) * nd) for s in out_shapes]
    out_shape = [jax.ShapeDtypeStruct(s, jnp.float32) for s in out_shapes]
    n_in = len(ins)

    def body(*refs):
        vals = fn(*[ref[...] for ref in refs[:n_in]])
        if not isinstance(vals, (tuple, list)):
            vals = (vals,)
        for oref, v in zip(refs[n_in:], vals):
            oref[...] = v

    res = pl.pallas_call(
        body, in_specs=in_specs, out_specs=out_specs, out_shape=out_shape,
    )(*ins)
    return res if len(out_shapes) > 1 else res[0]


def kernel(nxyz, nbr_list, num_atoms, embed, Wg1, bg1, Wg2, bg2, Win, bin,
           Wo1, bo1, Wo2, bo2, Wr1, br1, Wr2, br2, Wa, ba):
    f32 = jnp.float32
    sigma = float(CUTOFF / (G - 1))
    inv2s2 = 1.0 / (2.0 * sigma * sigma)

    def _mu_row():
        return lax.broadcasted_iota(jnp.int32, (1, G), 1).astype(f32) * (
            CUTOFF / (G - 1))

    z = nxyz[:, 0].astype(jnp.int32)
    xyz = nxyz[:, 1:4].astype(f32)
    a0 = nbr_list[:, 0].astype(jnp.int32)
    a1 = nbr_list[:, 1].astype(jnp.int32)

    zp = jnp.pad(z, (0, NP - N))
    xyzp = jnp.pad(xyz, ((0, NP - N), (0, 13)))
    a0r = a0.reshape(E // CF, CF)
    a1r = a1.reshape(E // CF, CF)
    a0g = a0.reshape(E // C, C)
    a1g = a1.reshape(E // C, C)
    zrowF = jnp.zeros((CF, D), f32)
    zrow16 = jnp.zeros((C, 16), f32)
    zrow128 = jnp.zeros((C, D), f32)

    r = _sc_gather(NP, 100, D)(embed, zp)
    x0, x1 = _sc_pair_gather(16)(xyzp, a0g, a1g)

    def geom_fn(x0b, x1b):
        df = x0b - x1b
        d = jnp.sqrt(jnp.sum(df * df, axis=1, keepdims=True) + 1e-12)
        gv = jnp.exp(-((d - _mu_row()) ** 2) * inv2s2)
        return d, gv

    dcol, gvec = _tc_map(geom_fn, [x0, x1], [], [1, G], E_BLK)

    b2 = lambda v: v.reshape(1, -1)
    Ws_l, h_l, u_l = [], [], []
    for i in range(CONVS):
        def filt_fn(gb, W1, b1, W2, bb2):
            return _ssp(jnp.dot(gb, W1, preferred_element_type=f32) + b1) @ W2 + bb2

        We = _tc_map(filt_fn, [gvec], [Wg1[i], b2(bg1[i]), Wg2[i], b2(bg2[i])],
                     [D], E_BLK)

        def h_fn(rb, W, bb):
            return jnp.dot(rb, W, preferred_element_type=f32) + bb

        h = _tc_map(h_fn, [r], [Win[i], b2(bin[i])], [D], N_BLK)
        aggp = _sc_conv_fwd()(h, We, a0r, a1r, zrowF)

        def out_fn(rb, g0b, g1b, W1, b1, W2, bb2):
            u = jnp.dot(g0b + g1b, W1, preferred_element_type=f32) + b1
            rn = rb + jnp.dot(_ssp(u), W2, preferred_element_type=f32) + bb2
            return u, rn

        u, r = _tc_map(out_fn, [r, aggp[0], aggp[1]],
                       [Wo1[i], b2(bo1[i]), Wo2[i], b2(bo2[i])], [D, D], N_BLK)
        Ws_l.append(We)
        h_l.append(h)
        u_l.append(u)

    Wa8 = jnp.pad(Wa, ((0, 0), (0, 0), (0, 5)))
    ba8 = jnp.pad(ba, ((0, 0), (0, 5)))
    ur_l, p_l, ae_l, pa_l = [], [], [], []
    for k in range(2):
        def ro_fn(rb, W1, b1, W2, bb2, Wap, bap):
            ur = jnp.dot(rb, W1, preferred_element_type=f32) + b1
            ae = jnp.dot(_ssp(ur), W2, preferred_element_type=f32) + bb2
            p = jnp.dot(rb, Wap, preferred_element_type=f32) + bap
            return ur, ae, p, _sp(p)

        ur, ae, p, pa = _tc_map(
            ro_fn, [r],
            [Wr1[k], b2(br1[k]), Wr2[k], b2(br2[k]), Wa8[k], b2(ba8[k])],
            [64, 1, 8, 8], N_BLK)
        ur_l.append(ur); p_l.append(p); ae_l.append(ae); pa_l.append(pa)

    patom = jnp.concatenate(
        [pa_l[0][:, :3], pa_l[1][:, :3], jnp.zeros((NP, 10), f32)], axis=1)
    P0, P1 = _sc_pair_gather(16)(patom, a0g, a1g)

    def eb_fn(p0b, p1b, db):
        cols = []
        d1 = db[:, 0]
        for k in range(2):
            Dp = 0.5 * (p0b[:, 3 * k] + p1b[:, 3 * k])
            ap = 0.5 * (p0b[:, 3 * k + 1] + p1b[:, 3 * k + 1])
            r0 = 0.5 * (p0b[:, 3 * k + 2] + p1b[:, 3 * k + 2])
            q = 1.0 - jnp.exp(-ap * (d1 - r0))
            cols.append((Dp * q * q)[:, None])
        return jnp.concatenate(cols + [jnp.zeros_like(p0b[:, :14])], axis=1)

    ebrow = _tc_map(eb_fn, [P0, P1, dcol], [], [16], E_BLK)
    ebp = _sc_scat16(False, False)(ebrow, a0g, a1g, zrow16)

    def mol_fn(ae0b, ae1b, e00, e01, e10, e11):
        m0 = jnp.sum(ae0b, axis=1) + jnp.sum(e00 + e01, axis=1)
        m1 = jnp.sum(ae1b, axis=1) + jnp.sum(e10 + e11, axis=1)
        se0 = jnp.minimum(m0, m1)
        se1 = jnp.maximum(m0, m1)
        t0 = (m0 <= m1).astype(f32)
        out = jnp.stack([se0, se1, t0, 1.0 - t0], axis=1)
        return jnp.pad(out, ((0, 0), (0, 4)))

    mol_ins = [ae_l[0][:N, 0].reshape(B, N // B), ae_l[1][:N, 0].reshape(B, N // B),
               ebp[0, :N, 0].reshape(B, N // B), ebp[1, :N, 0].reshape(B, N // B),
               ebp[0, :N, 1].reshape(B, N // B), ebp[1, :N, 1].reshape(B, N // B)]
    mol = _tc_whole(mol_fn, mol_ins, [(B, 8)])
    se0, se1 = mol[:, 0], mol[:, 1]
    w = mol[:, 2:4]
    seeds_at = jnp.repeat(
        jnp.concatenate([w, 1.0 - w], axis=1), N // B, axis=0)
    seeds_at = jnp.pad(seeds_at, ((0, NP - N), (0, 0)))

    patom2 = jnp.concatenate(
        [pa_l[0][:, :3], pa_l[1][:, :3], seeds_at, jnp.zeros((NP, 6), f32)],
        axis=1)
    P0b, P1b = _sc_pair_gather(16)(patom2, a0g, a1g)

    Wr2T = [b2(Wr2[k][:, 0]) for k in range(2)]
    Wr1T = [Wr1[k].T for k in range(2)]
    WaT8 = [jnp.pad(Wa[k].T, ((0, 5), (0, 0))) for k in range(2)]
    Wo2T = [Wo2[i].T for i in range(CONVS)]
    Wo1T = [Wo1[i].T for i in range(CONVS)]
    WinT = [Win[i].T for i in range(CONVS)]
    Wg2T = [Wg2[i].T for i in range(CONVS)]
    Wg1T = [Wg1[i].T for i in range(CONVS)]

    def backward(c):
        def ebb_fn(p0b, p1b, db):
            d1 = db[:, 0]
            dbar = jnp.zeros_like(d1)
            cols = []
            for k in range(2):
                seed = p0b[:, 6 + 2 * c + k]
                Dp = 0.5 * (p0b[:, 3 * k] + p1b[:, 3 * k])
                ap = 0.5 * (p0b[:, 3 * k + 1] + p1b[:, 3 * k + 1])
                r0 = 0.5 * (p0b[:, 3 * k + 2] + p1b[:, 3 * k + 2])
                ex = jnp.exp(-ap * (d1 - r0))
                q = 1.0 - ex
                Dpb = seed * q * q
                qb = seed * Dp * 2.0 * q
                apb = qb * ex * (d1 - r0)
                dbar = dbar + qb * ex * ap
                r0b = -qb * ex * ap
                cols += [(0.5 * Dpb)[:, None], (0.5 * apb)[:, None],
                         (0.5 * r0b)[:, None]]
            crow = jnp.concatenate(cols + [jnp.zeros_like(p0b[:, :10])], axis=1)
            return dbar[:, None], crow

        dbar_eb, crow = _tc_map(ebb_fn, [P0b, P1b, dcol], [], [1, 16], E_BLK)
        pbp = _sc_scat16(True, False)(crow, a0g, a1g, zrow16)

        def rbar_fn(pba, pbb, ur0, ur1, pp0, pp1, sa,
                    wr2t0, wr1t0, wat0, wr2t1, wr1t1, wat1):
            pb = pba + pbb
            rb = jnp.zeros((pba.shape[0], D), f32)
            for k, (wr2t, wr1t, wat, urk, ppk) in enumerate(
                    [(wr2t0, wr1t0, wat0, ur0, pp0),
                     (wr2t1, wr1t1, wat1, ur1, pp1)]):
                seed = sa[:, 2 * c + k:2 * c + k + 1]
                urb = (seed * wr2t) * _sig(urk)
                rb = rb + jnp.dot(urb, wr1t, preferred_element_type=f32)
                pbk = pb[:, 3 * k:3 * k + 3] * _sig(ppk[:, :3])
                pbk = jnp.concatenate(
                    [pbk, jnp.zeros_like(ppk[:, :5])], axis=1)
                rb = rb + jnp.dot(pbk, wat, preferred_element_type=f32)
            return rb

        sa_cols = jnp.concatenate([seeds_at, jnp.zeros((NP, 4), f32)], axis=1)
        rbar = _tc_map(
            rbar_fn,
            [pbp[0], pbp[1], ur_l[0], ur_l[1], p_l[0], p_l[1], sa_cols],
            [Wr2T[0], Wr1T[0], WaT8[0], Wr2T[1], Wr1T[1], WaT8[1]],
            [D], N_BLK)

        gbar = None
        for i in reversed(range(CONVS)):
            def aggb_fn(rbb, ub, w2t, w1t):
                vb = jnp.dot(rbb, w2t, preferred_element_type=f32)
                return jnp.dot(vb * _sig(ub), w1t, preferred_element_type=f32)

            aggbar = _tc_map(aggb_fn, [rbar, u_l[i]], [Wo2T[i], Wo1T[i]],
                             [D], N_BLK)
            Wbar, hbp = _sc_conv_bwd()(aggbar, h_l[i], Ws_l[i], a0r, a1r,
                                       zrowF)

            def rbup_fn(rbb, hb0, hb1, wt):
                return rbb + jnp.dot(hb0 + hb1, wt, preferred_element_type=f32)

            rbar = _tc_map(rbup_fn, [rbar, hbp[0], hbp[1]], [WinT[i]],
                           [D], N_BLK)

            def gb_fn(wbb, gvb, w2t, w1, b1, w1t, *gprev):
                sb = jnp.dot(wbb, w2t, preferred_element_type=f32)
                tb = sb * _sig(jnp.dot(gvb, w1, preferred_element_type=f32) + b1)
                out = jnp.dot(tb, w1t, preferred_element_type=f32)
                return out + gprev[0] if gprev else out

            gins = [Wbar, gvec] + ([gbar] if gbar is not None else [])

            def gb_wrap(wbb, gvb, *rest):
                auxs = rest[-4:]
                gp = rest[:-4]
                return gb_fn(wbb, gvb, auxs[0], auxs[1], auxs[2], auxs[3], *gp)

            gbar = _tc_map(gb_wrap, gins,
                           [Wg2T[i], Wg1[i], b2(bg1[i]), Wg1T[i]], [G], E_BLK)

        def delta_fn(dbe, gbb, gvb, db, x0b, x1b):
            d1 = db[:, 0:1]
            dbar = dbe + jnp.sum(
                gbb * gvb * (_mu_row() - d1), axis=1, keepdims=True) * (
                    1.0 / (sigma * sigma))
            coef = dbar / d1
            return coef * (x0b - x1b)

        delta = _tc_map(delta_fn, [dbar_eb, gbar, gvec, dcol, x0, x1],
                        [], [16], E_BLK)
        gp = _sc_scat16(True, True)(delta, a0g, a1g, zrow16)
        gsum = _tc_map(lambda a, bq, cq, dq: a + bq - cq - dq,
                       [gp[0, 0], gp[1, 0], gp[0, 1], gp[1, 1]], [],
                       [16], N_BLK)
        return gsum[:N, :3]

    g0 = backward(0)
    g1 = backward(1)
    return (se0, se1, g0, g1)


# speedup vs baseline: 6.2255x; 1.0248x over previous
import functools

import jax
import jax.numpy as jnp
from jax import lax
from jax.experimental import pallas as pl
from jax.experimental.pallas import tpu as pltpu
from jax.experimental.pallas import tpu_sc as plsc

N = 10000
E = 320000
B = 100
D = 128
G = 32
CONVS = 3
CUTOFF = 5.0

NP = 10240
NW = 32
C = 80
E_BLK = 3200
N_BLK = 2048

_LOG2 = 0.6931471805599453


def _ssp(x):
    return jnp.maximum(x, 0.0) + jnp.log1p(jnp.exp(-jnp.abs(x))) - _LOG2


def _sp(x):
    return jnp.maximum(x, 0.0) + jnp.log1p(jnp.exp(-jnp.abs(x)))


def _sig(x):
    return 1.0 / (1.0 + jnp.exp(-x))


def _mesh():
    return plsc.VectorSubcoreMesh(core_axis_name="c", subcore_axis_name="s")


@functools.lru_cache(maxsize=None)
def _sc_gather(n_idx, n_tab, dw):
    per_w = n_idx // NW
    nchunk = per_w // C

    @functools.partial(
        pl.kernel,
        mesh=_mesh(),
        out_type=jax.ShapeDtypeStruct((n_idx, dw), jnp.float32),
        compiler_params=pltpu.CompilerParams(use_tc_tiling_on_sc=False),
        scratch_types=[
            pltpu.VMEM((C,), jnp.int32),
            pltpu.VMEM((C, dw), jnp.float32),
            pltpu.SemaphoreType.DMA,
        ],
    )
    def k(table_hbm, idx_hbm, out_hbm, idx_v, rows_v, sem):
        wid = lax.axis_index("s") * 2 + lax.axis_index("c")
        base = wid * per_w

        def body(j, carry):
            off = base + j * C
            pltpu.sync_copy(idx_hbm.at[pl.ds(off, C)], idx_v)
            pltpu.async_copy(table_hbm.at[idx_v], rows_v, sem).wait()
            pltpu.sync_copy(rows_v, out_hbm.at[pl.ds(off, C)])
            return carry

        lax.fori_loop(0, nchunk, body, 0)

    return k


@functools.lru_cache(maxsize=None)
def _sc_scatter_add(n_vals, dw):
    per_w = n_vals // NW
    nchunk = per_w // C
    stripe = NP // 16

    @functools.partial(
        pl.kernel,
        mesh=_mesh(),
        out_type=jax.ShapeDtypeStruct((2, NP, dw), jnp.float32),
        compiler_params=pltpu.CompilerParams(use_tc_tiling_on_sc=False),
        scratch_types=[
            pltpu.VMEM((C,), jnp.int32),
            pltpu.VMEM((C, dw), jnp.float32),
            pltpu.VMEM_SHARED((NP, dw), jnp.float32),
            pltpu.SemaphoreType.DMA,
        ],
    )
    def k(vals_hbm, idx_hbm, zrow_hbm, out_hbm, idx_v, vals_v, acc_sh, sem):
        cid = lax.axis_index("c")
        sid = lax.axis_index("s")
        wid = sid * 2 + cid
        base = wid * per_w

        pltpu.sync_copy(zrow_hbm, vals_v)

        def zbody(j, carry):
            pltpu.sync_copy(vals_v, acc_sh.at[pl.ds(sid * stripe + j * C, C)])
            return carry

        lax.fori_loop(0, stripe // C, zbody, 0)
        plsc.subcore_barrier()

        def body(j, carry):
            off = base + j * C
            pltpu.sync_copy(idx_hbm.at[pl.ds(off, C)], idx_v)
            pltpu.sync_copy(vals_hbm.at[pl.ds(off, C)], vals_v)
            pltpu.sync_copy(vals_v, acc_sh.at[idx_v], add=True)
            return carry

        lax.fori_loop(0, nchunk, body, 0)
        plsc.subcore_barrier()
        pltpu.sync_copy(
            acc_sh.at[pl.ds(sid * stripe, stripe)],
            out_hbm.at[cid, pl.ds(sid * stripe, stripe)],
        )

    return k


CF = 40
_NCHF = (E // NW) // CF


@functools.lru_cache(maxsize=None)
def _sc_pair_gather(dw):
    per_w = E // NW
    nchunk = per_w // C

    @functools.partial(
        pl.kernel,
        mesh=_mesh(),
        out_type=(jax.ShapeDtypeStruct((E, dw), jnp.float32),
                  jax.ShapeDtypeStruct((E, dw), jnp.float32)),
        compiler_params=pltpu.CompilerParams(use_tc_tiling_on_sc=False),
        scratch_types=[
            pltpu.VMEM((nchunk, C), jnp.int32),
            pltpu.VMEM((nchunk, C), jnp.int32),
            pltpu.VMEM((2, C, dw), jnp.float32),
            pltpu.VMEM((2, C, dw), jnp.float32),
            pltpu.SemaphoreType.DMA,
            pltpu.SemaphoreType.DMA,
            pltpu.SemaphoreType.DMA,
            pltpu.SemaphoreType.DMA,
            pltpu.SemaphoreType.DMA,
            pltpu.SemaphoreType.DMA,
            pltpu.SemaphoreType.DMA,
            pltpu.SemaphoreType.DMA,
        ],
    )
    def k(tab_hbm, a0r_hbm, a1r_hbm, o0_hbm, o1_hbm,
          idx0_v, idx1_v, r0_v, r1_v,
          g00, g01, g10, g11, w00, w01, w10, w11):
        cid = lax.axis_index("c")
        sid = lax.axis_index("s")
        wid = sid * 2 + cid
        base = wid * per_w
        pltpu.sync_copy(a0r_hbm.at[pl.ds(wid * nchunk, nchunk)], idx0_v)
        pltpu.sync_copy(a1r_hbm.at[pl.ds(wid * nchunk, nchunk)], idx1_v)

        def start(j, b, ga, gb):
            c0 = pltpu.async_copy(tab_hbm.at[idx0_v.at[j]], r0_v.at[b], ga)
            c1 = pltpu.async_copy(tab_hbm.at[idx1_v.at[j]], r1_v.at[b], gb)
            return c0, c1

        def drain(j, b, c0, c1, wa, wb):
            c0.wait()
            c1.wait()
            s = pl.ds(base + j * C, C)
            w0 = pltpu.async_copy(r0_v.at[b], o0_hbm.at[s], wa)
            w1 = pltpu.async_copy(r1_v.at[b], o1_hbm.at[s], wb)
            return w0, w1

        def body(t, carry):
            j0 = 2 * t
            j1 = j0 + 1
            ca = start(j0, 0, g00, g01)
            cb = start(j1, 1, g10, g11)
            wa = drain(j0, 0, *ca, w00, w01)
            wb = drain(j1, 1, *cb, w10, w11)
            for w in (*wa, *wb):
                w.wait()
            return carry

        lax.fori_loop(0, nchunk // 2, body, 0)
        ca = start(nchunk - 1, 0, g00, g01)
        wa = drain(nchunk - 1, 0, *ca, w00, w01)
        for w in wa:
            w.wait()

    return k


@functools.lru_cache(maxsize=None)
def _sc_scat16(two_idx, two_acc):
    per_w = E // NW
    nchunk = per_w // C
    stripe = NP // 16
    dw = 16
    nacc = 2 if two_acc else 1
    oshape = (2, 2, NP, dw) if two_acc else (2, NP, dw)

    scratch = [
        pltpu.VMEM((nchunk, C), jnp.int32),
        pltpu.VMEM((nchunk, C), jnp.int32),
        pltpu.VMEM((2, C, dw), jnp.float32),
    ] + [pltpu.VMEM_SHARED((NP, dw), jnp.float32) for _ in range(nacc)] + [
        pltpu.SemaphoreType.DMA,
        pltpu.SemaphoreType.DMA,
    ]

    @functools.partial(
        pl.kernel,
        mesh=_mesh(),
        out_type=jax.ShapeDtypeStruct(oshape, jnp.float32),
        compiler_params=pltpu.CompilerParams(use_tc_tiling_on_sc=False),
        scratch_types=scratch,
    )
    def k(vals_hbm, a0r_hbm, a1r_hbm, zrow_hbm, out_hbm,
          idx0_v, idx1_v, vals_v, *rest):
        accs = rest[:nacc]
        s0, s1 = rest[nacc], rest[nacc + 1]
        cid = lax.axis_index("c")
        sid = lax.axis_index("s")
        wid = sid * 2 + cid
        base = wid * per_w

        pltpu.sync_copy(zrow_hbm, vals_v.at[0])

        def zbody(j, carry):
            for a in range(nacc):
                pltpu.sync_copy(
                    vals_v.at[0],
                    accs[a].at[pl.ds(sid * stripe + j * C, C)])
            return carry

        lax.fori_loop(0, stripe // C, zbody, 0)
        pltpu.sync_copy(a0r_hbm.at[pl.ds(wid * nchunk, nchunk)], idx0_v)
        if two_idx:
            pltpu.sync_copy(a1r_hbm.at[pl.ds(wid * nchunk, nchunk)], idx1_v)
        plsc.subcore_barrier()

        def start(j, b, sem):
            return pltpu.async_copy(
                vals_hbm.at[pl.ds(base + j * C, C)], vals_v.at[b], sem)

        def drain(j, b, cp):
            cp.wait()
            pltpu.sync_copy(vals_v.at[b], accs[0].at[idx0_v.at[j]], add=True)
            if two_idx:
                pltpu.sync_copy(vals_v.at[b], accs[nacc - 1].at[idx1_v.at[j]],
                                add=True)

        def body(t, carry):
            j0 = 2 * t
            j1 = j0 + 1
            c0 = start(j0, 0, s0)
            c1 = start(j1, 1, s1)
            drain(j0, 0, c0)
            drain(j1, 1, c1)
            return carry

        lax.fori_loop(0, nchunk // 2, body, 0)
        c0 = start(nchunk - 1, 0, s0)
        drain(nchunk - 1, 0, c0)
        plsc.subcore_barrier()
        for a in range(nacc):
            src = accs[a].at[pl.ds(sid * stripe, stripe)]
            if two_acc:
                pltpu.sync_copy(src, out_hbm.at[cid, a,
                                                pl.ds(sid * stripe, stripe)])
            else:
                pltpu.sync_copy(src, out_hbm.at[cid,
                                                pl.ds(sid * stripe, stripe)])

    return k


@functools.lru_cache(maxsize=None)
def _sc_conv_fwd():
    per_w = E // NW
    nchunk = _NCHF
    stripe = NP // 16

    @functools.partial(
        pl.kernel,
        mesh=_mesh(),
        out_type=jax.ShapeDtypeStruct((2, NP, D), jnp.float32),
        compiler_params=pltpu.CompilerParams(use_tc_tiling_on_sc=False),
        scratch_types=[
            pltpu.VMEM((nchunk, CF), jnp.int32),
            pltpu.VMEM((nchunk, CF), jnp.int32),
            pltpu.VMEM((2, CF, D), jnp.float32),
            pltpu.VMEM((2, CF, D), jnp.float32),
            pltpu.VMEM_SHARED((NP, D), jnp.float32),
            pltpu.SemaphoreType.DMA,
            pltpu.SemaphoreType.DMA,
            pltpu.SemaphoreType.DMA,
            pltpu.SemaphoreType.DMA,
        ],
    )
    def k(h_hbm, we_hbm, a0r_hbm, a1r_hbm, zrow_hbm, out_hbm,
          idx0_v, idx1_v, hrow_v, werow_v, acc_sh, g0s, g1s, w0s, w1s):
        cid = lax.axis_index("c")
        sid = lax.axis_index("s")
        wid = sid * 2 + cid
        base = wid * per_w

        pltpu.sync_copy(zrow_hbm, hrow_v.at[0])

        def zbody(j, carry):
            pltpu.sync_copy(hrow_v.at[0],
                            acc_sh.at[pl.ds(sid * stripe + j * CF, CF)])
            return carry

        lax.fori_loop(0, stripe // CF, zbody, 0)
        pltpu.sync_copy(a0r_hbm.at[pl.ds(wid * nchunk, nchunk)], idx0_v)
        pltpu.sync_copy(a1r_hbm.at[pl.ds(wid * nchunk, nchunk)], idx1_v)
        plsc.subcore_barrier()

        def start(j, b, gs, ws):
            cg = pltpu.async_copy(h_hbm.at[idx1_v.at[j]], hrow_v.at[b], gs)
            cw = pltpu.async_copy(we_hbm.at[pl.ds(base + j * CF, CF)],
                                  werow_v.at[b], ws)
            return cg, cw

        def compute(j, b, cg, cw):
            cg.wait()
            cw.wait()

            def mul(i, carry2):
                for v in range(D // 16):
                    s = pl.ds(v * 16, 16)
                    hrow_v[b, i, s] = hrow_v[b, i, s] * werow_v[b, i, s]
                return carry2

            lax.fori_loop(0, CF, mul, 0)
            pltpu.sync_copy(hrow_v.at[b], acc_sh.at[idx0_v.at[j]], add=True)

        def body(t, carry):
            j0 = 2 * t
            j1 = j0 + 1
            cg0, cw0 = start(j0, 0, g0s, w0s)
            cg1, cw1 = start(j1, 1, g1s, w1s)
            compute(j0, 0, cg0, cw0)
            compute(j1, 1, cg1, cw1)
            return carry

        lax.fori_loop(0, nchunk // 2, body, 0)
        plsc.subcore_barrier()
        pltpu.sync_copy(
            acc_sh.at[pl.ds(sid * stripe, stripe)],
            out_hbm.at[cid, pl.ds(sid * stripe, stripe)],
        )

    return k


@functools.lru_cache(maxsize=None)
def _sc_conv_bwd():
    per_w = E // NW
    nchunk = _NCHF
    stripe = NP // 16

    @functools.partial(
        pl.kernel,
        mesh=_mesh(),
        out_type=(jax.ShapeDtypeStruct((E, D), jnp.float32),
                  jax.ShapeDtypeStruct((2, NP, D), jnp.float32)),
        compiler_params=pltpu.CompilerParams(use_tc_tiling_on_sc=False),
        scratch_types=[
            pltpu.VMEM((nchunk, CF), jnp.int32),
            pltpu.VMEM((nchunk, CF), jnp.int32),
            pltpu.VMEM((2, CF, D), jnp.float32),
            pltpu.VMEM((2, CF, D), jnp.float32),
            pltpu.VMEM((CF, D), jnp.float32),
            pltpu.VMEM_SHARED((NP, D), jnp.float32),
            pltpu.SemaphoreType.DMA,
            pltpu.SemaphoreType.DMA,
            pltpu.SemaphoreType.DMA,
            pltpu.SemaphoreType.DMA,
        ],
    )
    def k(ab_hbm, h_hbm, we_hbm, a0r_hbm, a1r_hbm, zrow_hbm,
          wbar_hbm, out_hbm, idx0_v, idx1_v, mbrow_v, hrow_v, werow_v,
          acc_sh, m0s, m1s, g0s, g1s):
        cid = lax.axis_index("c")
        sid = lax.axis_index("s")
        wid = sid * 2 + cid
        base = wid * per_w

        pltpu.sync_copy(zrow_hbm, werow_v)

        def zbody(j, carry):
            pltpu.sync_copy(werow_v,
                            acc_sh.at[pl.ds(sid * stripe + j * CF, CF)])
            return carry

        lax.fori_loop(0, stripe // CF, zbody, 0)
        pltpu.sync_copy(a0r_hbm.at[pl.ds(wid * nchunk, nchunk)], idx0_v)
        pltpu.sync_copy(a1r_hbm.at[pl.ds(wid * nchunk, nchunk)], idx1_v)
        plsc.subcore_barrier()

        def start(j, b, ms, gs):
            cm = pltpu.async_copy(ab_hbm.at[idx0_v.at[j]], mbrow_v.at[b], ms)
            cg = pltpu.async_copy(h_hbm.at[idx1_v.at[j]], hrow_v.at[b], gs)
            return cm, cg

        def compute(j, b, cm, cg):
            pltpu.sync_copy(we_hbm.at[pl.ds(base + j * CF, CF)], werow_v)
            cm.wait()
            cg.wait()

            def mul(i, carry2):
                for v in range(D // 16):
                    s = pl.ds(v * 16, 16)
                    mb = mbrow_v[b, i, s]
                    hrow_v[b, i, s] = mb * hrow_v[b, i, s]
                    werow_v[i, s] = mb * werow_v[i, s]
                return carry2

            lax.fori_loop(0, CF, mul, 0)
            pltpu.sync_copy(hrow_v.at[b], wbar_hbm.at[pl.ds(base + j * CF, CF)])
            pltpu.sync_copy(werow_v, acc_sh.at[idx1_v.at[j]], add=True)

        def body(t, carry):
            j0 = 2 * t
            j1 = j0 + 1
            c0 = start(j0, 0, m0s, g0s)
            c1 = start(j1, 1, m1s, g1s)
            compute(j0, 0, *c0)
            compute(j1, 1, *c1)
            return carry

        lax.fori_loop(0, nchunk // 2, body, 0)
        plsc.subcore_barrier()
        pltpu.sync_copy(
            acc_sh.at[pl.ds(sid * stripe, stripe)],
            out_hbm.at[cid, pl.ds(sid * stripe, stripe)],
        )

    return k


def _tc_map(fn, rows, auxs, out_cols, blk):
    R = rows[0].shape[0]
    grid = R // blk

    def row_spec(x):
        nd = x.ndim
        bs = (blk,) + x.shape[1:]
        return pl.BlockSpec(bs, lambda i, nd=nd: (i,) + (0,) * (nd - 1))

    def aux_spec(a):
        nd = a.ndim
        return pl.BlockSpec(a.shape, lambda i, nd=nd: (0,) * nd)

    in_specs = [row_spec(x) for x in rows] + [aux_spec(a) for a in auxs]
    out_specs = [pl.BlockSpec((blk, c), lambda i: (i, 0)) for c in out_cols]
    out_shape = [jax.ShapeDtypeStruct((R, c), jnp.float32) for c in out_cols]
    n_in = len(rows) + len(auxs)

    def body(*refs):
        vals = fn(*[ref[...] for ref in refs[:n_in]])
        if not isinstance(vals, (tuple, list)):
            vals = (vals,)
        for oref, v in zip(refs[n_in:], vals):
            oref[...] = v

    res = pl.pallas_call(
        body, grid=(grid,), in_specs=in_specs, out_specs=out_specs,
        out_shape=out_shape,
    )(*rows, *auxs)
    return res if len(out_cols) > 1 else res[0]


def _tc_whole(fn, ins, out_shapes):
    in_specs = [pl.BlockSpec(x.shape, lambda *_, nd=x.ndim: (0,) * nd) for x in ins]
    out_specs = [pl.BlockSpec(s, lambda *_, nd=len(s): (0,) * nd) for s in out_shapes]
    out_shape = [jax.ShapeDtypeStruct(s, jnp.float32) for s in out_shapes]
    n_in = len(ins)

    def body(*refs):
        vals = fn(*[ref[...] for ref in refs[:n_in]])
        if not isinstance(vals, (tuple, list)):
            vals = (vals,)
        for oref, v in zip(refs[n_in:], vals):
            oref[...] = v

    res = pl.pallas_call(
        body, in_specs=in_specs, out_specs=out_specs, out_shape=out_shape,
    )(*ins)
    return res if len(out_shapes) > 1 else res[0]


def kernel(nxyz, nbr_list, num_atoms, embed, Wg1, bg1, Wg2, bg2, Win, bin,
           Wo1, bo1, Wo2, bo2, Wr1, br1, Wr2, br2, Wa, ba):
    f32 = jnp.float32
    sigma = float(CUTOFF / (G - 1))
    inv2s2 = 1.0 / (2.0 * sigma * sigma)

    def _mu_row():
        return lax.broadcasted_iota(jnp.int32, (1, G), 1).astype(f32) * (
            CUTOFF / (G - 1))

    z = nxyz[:, 0].astype(jnp.int32)
    xyz = nxyz[:, 1:4].astype(f32)
    a0 = nbr_list[:, 0].astype(jnp.int32)
    a1 = nbr_list[:, 1].astype(jnp.int32)

    zp = jnp.pad(z, (0, NP - N))
    xyzp = jnp.pad(xyz, ((0, NP - N), (0, 13)))
    a0r = a0.reshape(E // CF, CF)
    a1r = a1.reshape(E // CF, CF)
    a0g = a0.reshape(E // C, C)
    a1g = a1.reshape(E // C, C)
    zrowF = jnp.zeros((CF, D), f32)
    zrow16 = jnp.zeros((C, 16), f32)
    zrow128 = jnp.zeros((C, D), f32)

    r = _sc_gather(NP, 100, D)(embed, zp)
    x0, x1 = _sc_pair_gather(16)(xyzp, a0g, a1g)

    def geom_fn(x0b, x1b):
        df = x0b - x1b
        d = jnp.sqrt(jnp.sum(df * df, axis=1, keepdims=True) + 1e-12)
        gv = jnp.exp(-((d - _mu_row()) ** 2) * inv2s2)
        return d, gv

    dcol, gvec = _tc_map(geom_fn, [x0, x1], [], [1, G], E_BLK)

    b2 = lambda v: v.reshape(1, -1)
    Ws_l, h_l, u_l = [], [], []
    for i in range(CONVS):
        def filt_fn(gb, W1, b1, W2, bb2):
            return _ssp(jnp.dot(gb, W1, preferred_element_type=f32) + b1) @ W2 + bb2

        We = _tc_map(filt_fn, [gvec], [Wg1[i], b2(bg1[i]), Wg2[i], b2(bg2[i])],
                     [D], E_BLK)

        def h_fn(rb, W, bb):
            return jnp.dot(rb, W, preferred_element_type=f32) + bb

        h = _tc_map(h_fn, [r], [Win[i], b2(bin[i])], [D], N_BLK)
        aggp = _sc_conv_fwd()(h, We, a0r, a1r, zrowF)

        def out_fn(rb, g0b, g1b, W1, b1, W2, bb2):
            u = jnp.dot(g0b + g1b, W1, preferred_element_type=f32) + b1
            rn = rb + jnp.dot(_ssp(u), W2, preferred_element_type=f32) + bb2
            return u, rn

        u, r = _tc_map(out_fn, [r, aggp[0], aggp[1]],
                       [Wo1[i], b2(bo1[i]), Wo2[i], b2(bo2[i])], [D, D], N_BLK)
        Ws_l.append(We)
        h_l.append(h)
        u_l.append(u)

    Wa8 = jnp.pad(Wa, ((0, 0), (0, 0), (0, 5)))
    ba8 = jnp.pad(ba, ((0, 0), (0, 5)))
    ur_l, p_l, ae_l, pa_l = [], [], [], []
    for k in range(2):
        def ro_fn(rb, W1, b1, W2, bb2, Wap, bap):
            ur = jnp.dot(rb, W1, preferred_element_type=f32) + b1
            ae = jnp.dot(_ssp(ur), W2, preferred_element_type=f32) + bb2
            p = jnp.dot(rb, Wap, preferred_element_type=f32) + bap
            return ur, ae, p, _sp(p)

        ur, ae, p, pa = _tc_map(
            ro_fn, [r],
            [Wr1[k], b2(br1[k]), Wr2[k], b2(br2[k]), Wa8[k], b2(ba8[k])],
            [64, 1, 8, 8], N_BLK)
        ur_l.append(ur); p_l.append(p); ae_l.append(ae); pa_l.append(pa)

    patom = jnp.concatenate(
        [pa_l[0][:, :3], pa_l[1][:, :3], jnp.zeros((NP, 10), f32)], axis=1)
    P0, P1 = _sc_pair_gather(16)(patom, a0g, a1g)

    def eb_fn(p0b, p1b, db):
        cols = []
        d1 = db[:, 0]
        for k in range(2):
            Dp = 0.5 * (p0b[:, 3 * k] + p1b[:, 3 * k])
            ap = 0.5 * (p0b[:, 3 * k + 1] + p1b[:, 3 * k + 1])
            r0 = 0.5 * (p0b[:, 3 * k + 2] + p1b[:, 3 * k + 2])
            q = 1.0 - jnp.exp(-ap * (d1 - r0))
            cols.append((Dp * q * q)[:, None])
        return jnp.concatenate(cols + [jnp.zeros_like(p0b[:, :14])], axis=1)

    ebrow = _tc_map(eb_fn, [P0, P1, dcol], [], [16], E_BLK)
    ebp = _sc_scat16(False, False)(ebrow, a0g, a1g, zrow16)

    def mol_fn(ae0b, ae1b, e00, e01, e10, e11):
        m0 = jnp.sum(ae0b, axis=1) + jnp.sum(e00 + e01, axis=1)
        m1 = jnp.sum(ae1b, axis=1) + jnp.sum(e10 + e11, axis=1)
        se0 = jnp.minimum(m0, m1)
        se1 = jnp.maximum(m0, m1)
        t0 = (m0 <= m1).astype(f32)
        out = jnp.stack([se0, se1, t0, 1.0 - t0], axis=1)
        return jnp.pad(out, ((0, 0), (0, 4)))

    mol_ins = [ae_l[0][:N, 0].reshape(B, N // B), ae_l[1][:N, 0].reshape(B, N // B),
               ebp[0, :N, 0].reshape(B, N // B), ebp[1, :N, 0].reshape(B, N // B),
               ebp[0, :N, 1].reshape(B, N // B), ebp[1, :N, 1].reshape(B, N // B)]
    mol = _tc_whole(mol_fn, mol_ins, [(B, 8)])
    se0, se1 = mol[:, 0], mol[:, 1]
    w = mol[:, 2:4]
    seeds_at = jnp.repeat(
        jnp.concatenate([w, 1.0 - w], axis=1), N // B, axis=0)
    seeds_at = jnp.pad(seeds_at, ((0, NP - N), (0, 0)))

    patom2 = jnp.concatenate(
        [pa_l[0][:, :3], pa_l[1][:, :3], seeds_at, jnp.zeros((NP, 6), f32)],
        axis=1)
    P0b, P1b = _sc_pair_gather(16)(patom2, a0g, a1g)

    Wr2T = [b2(Wr2[k][:, 0]) for k in range(2)]
    Wr1T = [Wr1[k].T for k in range(2)]
    WaT8 = [jnp.pad(Wa[k].T, ((0, 5), (0, 0))) for k in range(2)]
    Wo2T = [Wo2[i].T for i in range(CONVS)]
    Wo1T = [Wo1[i].T for i in range(CONVS)]
    WinT = [Win[i].T for i in range(CONVS)]
    Wg2T = [Wg2[i].T for i in range(CONVS)]
    Wg1T = [Wg1[i].T for i in range(CONVS)]

    def ebb_fn(p0b, p1b, db):
        d1 = db[:, 0]
        outs = []
        for c in range(2):
            dbar = jnp.zeros_like(d1)
            cols = []
            for k in range(2):
                seed = p0b[:, 6 + 2 * c + k]
                Dp = 0.5 * (p0b[:, 3 * k] + p1b[:, 3 * k])
                ap = 0.5 * (p0b[:, 3 * k + 1] + p1b[:, 3 * k + 1])
                r0 = 0.5 * (p0b[:, 3 * k + 2] + p1b[:, 3 * k + 2])
                ex = jnp.exp(-ap * (d1 - r0))
                q = 1.0 - ex
                Dpb = seed * q * q
                qb = seed * Dp * 2.0 * q
                apb = qb * ex * (d1 - r0)
                dbar = dbar + qb * ex * ap
                r0b = -qb * ex * ap
                cols += [(0.5 * Dpb)[:, None], (0.5 * apb)[:, None],
                         (0.5 * r0b)[:, None]]
            crow = jnp.concatenate(
                cols + [jnp.zeros_like(p0b[:, :10])], axis=1)
            outs += [dbar[:, None], crow]
        return tuple(outs)

    dbar_eb0, crow0, dbar_eb1, crow1 = _tc_map(
        ebb_fn, [P0b, P1b, dcol], [], [1, 16, 1, 16], E_BLK)

    def backward(c, dbar_eb, crow):
        pbp = _sc_scat16(True, False)(crow, a0g, a1g, zrow16)

        def rbar_fn(pba, pbb, ur0, ur1, pp0, pp1, sa,
                    wr2t0, wr1t0, wat0, wr2t1, wr1t1, wat1):
            pb = pba + pbb
            rb = jnp.zeros((pba.shape[0], D), f32)
            for k, (wr2t, wr1t, wat, urk, ppk) in enumerate(
                    [(wr2t0, wr1t0, wat0, ur0, pp0),
                     (wr2t1, wr1t1, wat1, ur1, pp1)]):
                seed = sa[:, 2 * c + k:2 * c + k + 1]
                urb = (seed * wr2t) * _sig(urk)
                rb = rb + jnp.dot(urb, wr1t, preferred_element_type=f32)
                pbk = pb[:, 3 * k:3 * k + 3] * _sig(ppk[:, :3])
                pbk = jnp.concatenate(
                    [pbk, jnp.zeros_like(ppk[:, :5])], axis=1)
                rb = rb + jnp.dot(pbk, wat, preferred_element_type=f32)
            return rb

        sa_cols = jnp.concatenate([seeds_at, jnp.zeros((NP, 4), f32)], axis=1)
        rbar = _tc_map(
            rbar_fn,
            [pbp[0], pbp[1], ur_l[0], ur_l[1], p_l[0], p_l[1], sa_cols],
            [Wr2T[0], Wr1T[0], WaT8[0], Wr2T[1], Wr1T[1], WaT8[1]],
            [D], N_BLK)

        gbar = None

        def aggb_fn(rbb, ub, w2t, w1t):
            vb = jnp.dot(rbb, w2t, preferred_element_type=f32)
            return jnp.dot(vb * _sig(ub), w1t, preferred_element_type=f32)

        aggbar = _tc_map(aggb_fn, [rbar, u_l[CONVS - 1]],
                         [Wo2T[CONVS - 1], Wo1T[CONVS - 1]], [D], N_BLK)
        for i in reversed(range(CONVS)):
            Wbar, hbp = _sc_conv_bwd()(aggbar, h_l[i], Ws_l[i], a0r, a1r,
                                       zrowF)
            if i > 0:
                def step_fn(rbb, hb0, hb1, ub, wt, w2t, w1t):
                    rb = rbb + jnp.dot(hb0 + hb1, wt,
                                       preferred_element_type=f32)
                    vb = jnp.dot(rb, w2t, preferred_element_type=f32)
                    ag = jnp.dot(vb * _sig(ub), w1t,
                                 preferred_element_type=f32)
                    return rb, ag

                rbar, aggbar = _tc_map(
                    step_fn, [rbar, hbp[0], hbp[1], u_l[i - 1]],
                    [WinT[i], Wo2T[i - 1], Wo1T[i - 1]], [D, D], N_BLK)

            def gb_fn(wbb, gvb, w2t, w1, b1, w1t, *gprev):
                sb = jnp.dot(wbb, w2t, preferred_element_type=f32)
                tb = sb * _sig(jnp.dot(gvb, w1, preferred_element_type=f32) + b1)
                out = jnp.dot(tb, w1t, preferred_element_type=f32)
                return out + gprev[0] if gprev else out

            gins = [Wbar, gvec] + ([gbar] if gbar is not None else [])

            def gb_wrap(wbb, gvb, *rest):
                auxs = rest[-4:]
                gp = rest[:-4]
                return gb_fn(wbb, gvb, auxs[0], auxs[1], auxs[2], auxs[3], *gp)

            gbar = _tc_map(gb_wrap, gins,
                           [Wg2T[i], Wg1[i], b2(bg1[i]), Wg1T[i]], [G], E_BLK)

        def delta_fn(dbe, gbb, gvb, db, x0b, x1b):
            d1 = db[:, 0:1]
            dbar = dbe + jnp.sum(
                gbb * gvb * (_mu_row() - d1), axis=1, keepdims=True) * (
                    1.0 / (sigma * sigma))
            coef = dbar / d1
            return coef * (x0b - x1b)

        delta = _tc_map(delta_fn, [dbar_eb, gbar, gvec, dcol, x0, x1],
                        [], [16], E_BLK)
        gp = _sc_scat16(True, True)(delta, a0g, a1g, zrow16)
        gsum = _tc_map(lambda a, bq, cq, dq: a + bq - cq - dq,
                       [gp[0, 0], gp[1, 0], gp[0, 1], gp[1, 1]], [],
                       [16], N_BLK)
        return gsum[:N, :3]

    g0 = backward(0, dbar_eb0, crow0)
    g1 = backward(1, dbar_eb1, crow1)
    return (se0, se1, g0, g1)
